# Initial kernel scaffold; baseline (speedup 1.0000x reference)
import functools

import jax
import jax.numpy as jnp
from jax import lax
from jax.experimental import pallas as pl
from jax.experimental.pallas import tpu as pltpu
from jax.experimental.pallas import tpu_sc as plsc

_NC = 2
_NS = 16
_NW = _NC * _NS
_C = 80
_EPS = 1e-5
_F32 = jnp.float32


def _relu(v):
    return jnp.maximum(v, 0.0)


def _sigmoid(v):
    return 1.0 / (1.0 + jnp.exp(-v))


def _dot(a, b):
    return jnp.dot(a, b, preferred_element_type=jnp.float32)


def _node_stage1_body(x_ref, wh_ref, bh_ref, u_ref, bu_ref, v_ref, bv_ref,
                      b_ref, bb_ref, c_ref, bc_ref,
                      h0_ref, uh_ref, vc_ref, bh_out_ref):
    x = x_ref[...]
    h0 = _relu(x[:, 0:1] * wh_ref[0:1, :] + x[:, 1:2] * wh_ref[1:2, :]
               + bh_ref[0:1, :])
    h0_ref[...] = h0
    uh_ref[...] = _dot(h0, u_ref[...]) + bu_ref[0:1, :]
    vc_ref[:, 0:128] = _dot(h0, v_ref[...]) + bv_ref[0:1, :]
    vc_ref[:, 128:256] = _dot(h0, c_ref[...]) + bc_ref[0:1, :]
    bh_out_ref[...] = _dot(h0, b_ref[...]) + bb_ref[0:1, :]


def _node_stage2_body(uh_ref, agg2_ref, cnt2_ref, h0_ref, g_ref, b_ref,
                      b2_ref, bb2_ref, c2_ref, bc2_ref,
                      b2h_ref, c2h_ref):
    agg = agg2_ref[0] + agg2_ref[1]
    cnt = cnt2_ref[0, :, 0:1] + cnt2_ref[1, :, 0:1]
    q = uh_ref[...] + agg / jnp.maximum(cnt, 1.0)
    m = jnp.mean(q, axis=0, keepdims=True)
    v = jnp.mean((q - m) ** 2, axis=0, keepdims=True)
    h1 = h0_ref[...] + _relu((q - m) * lax.rsqrt(v + _EPS) * g_ref[0:1, :]
                             + b_ref[0:1, :])
    b2h_ref[...] = _dot(h1, b2_ref[...]) + bb2_ref[0:1, :]
    c2h_ref[...] = _dot(h1, c2_ref[...]) + bc2_ref[0:1, :]


def _e0_block(ea, wp_ref):
    return _relu(ea * wp_ref[0:1, :] + wp_ref[1:2, :])


def _epass1_l1_body(ea_ref, g1_ref, wp_ref, a1_ref, ba1_ref, sums_ref):
    i = pl.program_id(0)
    e0 = _e0_block(ea_ref[...], wp_ref)
    y = _dot(e0, a1_ref[...]) + ba1_ref[0:1, :] + g1_ref[...]

    @pl.when(i == 0)
    def _():
        sums_ref[...] = jnp.zeros_like(sums_ref)

    sums_ref[0:1, :] += jnp.sum(y, axis=0, keepdims=True)
    sums_ref[1:2, :] += jnp.sum(y * y, axis=0, keepdims=True)


def _e1_block(ea, g1, wp_ref, a1_ref, ba1_ref, sums1_ref, bn1_ref, inv_e):
    e0 = _e0_block(ea, wp_ref)
    y1 = _dot(e0, a1_ref[...]) + ba1_ref[0:1, :] + g1
    m1 = sums1_ref[0:1, :] * inv_e
    v1 = sums1_ref[1:2, :] * inv_e - m1 * m1
    return e0 + _relu((y1 - m1) * lax.rsqrt(v1 + _EPS) * bn1_ref[0:1, :]
                      + bn1_ref[1:2, :])


def _epass1_l2_body(ea_ref, g1_ref, g2_ref, wp_ref, a1_ref, ba1_ref,
                    sums1_ref, bn1_ref, a2_ref, ba2_ref, sums2_ref, *,
                    inv_e):
    i = pl.program_id(0)
    e1 = _e1_block(ea_ref[...], g1_ref[...], wp_ref, a1_ref, ba1_ref,
                   sums1_ref, bn1_ref, inv_e)
    y2 = _dot(e1, a2_ref[...]) + ba2_ref[0:1, :] + g2_ref[...]

    @pl.when(i == 0)
    def _():
        sums2_ref[...] = jnp.zeros_like(sums2_ref)

    sums2_ref[0:1, :] += jnp.sum(y2, axis=0, keepdims=True)
    sums2_ref[1:2, :] += jnp.sum(y2 * y2, axis=0, keepdims=True)


def _final_body(ea_ref, g1_ref, g2_ref, wp_ref, a1_ref, ba1_ref, sums1_ref,
                bn1_ref, a2_ref, ba2_ref, sums2_ref, bn2_ref,
                w1_ref, b1_ref, w2_ref, b2_ref, w3_ref, b3_ref, z_ref, *,
                inv_e):
    e1 = _e1_block(ea_ref[...], g1_ref[...], wp_ref, a1_ref, ba1_ref,
                   sums1_ref, bn1_ref, inv_e)
    y2 = _dot(e1, a2_ref[...]) + ba2_ref[0:1, :] + g2_ref[...]
    m2 = sums2_ref[0:1, :] * inv_e
    v2 = sums2_ref[1:2, :] * inv_e - m2 * m2
    e2 = e1 + _relu((y2 - m2) * lax.rsqrt(v2 + _EPS) * bn2_ref[0:1, :]
                    + bn2_ref[1:2, :])
    t = _dot(e2, w1_ref[...]) + b1_ref[0:1, :]
    t = t * _sigmoid(t)
    t = _dot(t, w2_ref[...]) + b2_ref[0:1, :]
    t = t * _sigmoid(t)
    z_ref[...] = _sigmoid(_dot(t, w3_ref[...]) + b3_ref[0:1, :])


def _sc_layer1_body(ea_hbm, src_hbm, dst_hbm, vc_hbm, bh_hbm, wp_hbm,
                    zero_nd_hbm, zero_n16_hbm,
                    g_out, agg_out, cnt_out,
                    src_v, dst_v, vcrows, brows, msgs, eav, wp_v, ones_v,
                    agg_sh, cnt_sh, sem1, sem2, *, n_nodes, ew, nchunk):
    c = lax.axis_index("c")
    s = lax.axis_index("s")
    base = (c * _NS + s) * ew
    rows = n_nodes // _NS
    r0 = s * rows

    pltpu.sync_copy(wp_hbm, wp_v)
    pltpu.sync_copy(zero_nd_hbm.at[pl.ds(r0, rows)], agg_sh.at[pl.ds(r0, rows)])
    pltpu.sync_copy(zero_n16_hbm.at[pl.ds(r0, rows)], cnt_sh.at[pl.ds(r0, rows)])
    for j in range(_C):
        ones_v[j, :] = jnp.full((16,), 1.0, _F32)
    plsc.subcore_barrier()

    def chunk_body(ci, carry):
        b0 = pl.multiple_of(base + ci * _C, 8)
        pltpu.sync_copy(src_hbm.at[pl.ds(b0, _C)], src_v)
        pltpu.sync_copy(dst_hbm.at[pl.ds(b0, _C)], dst_v)
        pltpu.sync_copy(ea_hbm.at[pl.ds(b0, _C)], eav)
        cp1 = pltpu.async_copy(vc_hbm.at[dst_v], vcrows, sem1)
        cp2 = pltpu.async_copy(bh_hbm.at[src_v], brows, sem2)
        cp1.wait()
        cp2.wait()

        def edge_body(j, carry2):
            a = eav[j]
            for k in range(8):
                sl = pl.ds(k * 16, 16)
                e0 = _relu(a * wp_v[0, sl] + wp_v[1, sl])
                msgs[j, sl] = _sigmoid(e0) * vcrows[j, sl]
                brows[j, sl] = brows[j, sl] + vcrows[j, pl.ds(128 + k * 16, 16)]
            return carry2

        lax.fori_loop(0, _C, edge_body, 0)
        pltpu.sync_copy(brows, g_out.at[pl.ds(b0, _C)])
        pltpu.sync_copy(msgs, agg_sh.at[src_v], add=True)
        pltpu.sync_copy(ones_v, cnt_sh.at[src_v], add=True)
        return carry

    lax.fori_loop(0, nchunk, chunk_body, 0)
    plsc.subcore_barrier()
    pltpu.sync_copy(agg_sh.at[pl.ds(r0, rows)], agg_out.at[c, pl.ds(r0, rows)])
    pltpu.sync_copy(cnt_sh.at[pl.ds(r0, rows)], cnt_out.at[c, pl.ds(r0, rows)])


def _sc_layer2_body(src_hbm, dst_hbm, b2_hbm, c2_hbm, g_out,
                    src_v, dst_v, brows, crows, sem1, sem2, *, ew, nchunk):
    c = lax.axis_index("c")
    s = lax.axis_index("s")
    base = (c * _NS + s) * ew

    def chunk_body(ci, carry):
        b0 = pl.multiple_of(base + ci * _C, 8)
        pltpu.sync_copy(src_hbm.at[pl.ds(b0, _C)], src_v)
        pltpu.sync_copy(dst_hbm.at[pl.ds(b0, _C)], dst_v)
        cp1 = pltpu.async_copy(b2_hbm.at[src_v], brows, sem1)
        cp2 = pltpu.async_copy(c2_hbm.at[dst_v], crows, sem2)
        cp1.wait()
        cp2.wait()

        def edge_body(j, carry2):
            for k in range(8):
                sl = pl.ds(k * 16, 16)
                brows[j, sl] = brows[j, sl] + crows[j, sl]
            return carry2

        lax.fori_loop(0, _C, edge_body, 0)
        pltpu.sync_copy(brows, g_out.at[pl.ds(b0, _C)])
        return carry

    lax.fori_loop(0, nchunk, chunk_body, 0)


def _full_spec(shape):
    return pl.BlockSpec(shape, lambda i: tuple(0 for _ in shape))


def _row2(w, b):
    return jnp.stack([w.reshape(-1), b.reshape(-1)], axis=0)


def kernel(x, edge_attr, edge_index, params):
    n = x.shape[0]
    e = edge_attr.shape[0]
    d = 128
    assert e % (_NW * _C) == 0 and n % _NS == 0
    ew = e // _NW
    nchunk = ew // _C
    be = 4000
    grid_e = e // be
    inv_e = 1.0 / e

    src = edge_index[0]
    dst = edge_index[1]
    ea = edge_attr.reshape(e)
    p = params
    l1, l2 = p["layers"][0], p["layers"][1]
    mlp = p["mlp"]

    f32 = jnp.float32
    sds = jax.ShapeDtypeStruct

    node1 = pl.pallas_call(
        _node_stage1_body,
        grid=(1,),
        in_specs=[_full_spec((n, 2))] + [_full_spec(s) for s in
                  [(2, d), (1, d), (d, d), (1, d), (d, d), (1, d),
                   (d, d), (1, d), (d, d), (1, d)]],
        out_specs=[_full_spec((n, d)), _full_spec((n, d)),
                   _full_spec((n, 2 * d)), _full_spec((n, d))],
        out_shape=[sds((n, d), f32), sds((n, d), f32),
                   sds((n, 2 * d), f32), sds((n, d), f32)],
    )
    h0, u1h, vc1, b1h = node1(
        x, p["h_proj"]["W"], p["h_proj"]["b"].reshape(1, d),
        l1["U"]["W"], l1["U"]["b"].reshape(1, d),
        l1["V"]["W"], l1["V"]["b"].reshape(1, d),
        l1["B"]["W"], l1["B"]["b"].reshape(1, d),
        l1["C"]["W"], l1["C"]["b"].reshape(1, d))

    wp2 = _row2(p["e_proj"]["W"], p["e_proj"]["b"])
    mesh = plsc.VectorSubcoreMesh(core_axis_name="c", subcore_axis_name="s")
    sc1 = pl.kernel(
        functools.partial(_sc_layer1_body, n_nodes=n, ew=ew, nchunk=nchunk),
        out_type=(sds((e, d), f32), sds((2, n, d), f32), sds((2, n, 16), f32)),
        mesh=mesh,
        scratch_types=[
            pltpu.VMEM((_C,), jnp.int32),
            pltpu.VMEM((_C,), jnp.int32),
            pltpu.VMEM((_C, 2 * d), f32),
            pltpu.VMEM((_C, d), f32),
            pltpu.VMEM((_C, d), f32),
            pltpu.VMEM((_C,), f32),
            pltpu.VMEM((2, d), f32),
            pltpu.VMEM((_C, 16), f32),
            pltpu.VMEM_SHARED((n, d), f32),
            pltpu.VMEM_SHARED((n, 16), f32),
            pltpu.SemaphoreType.DMA,
            pltpu.SemaphoreType.DMA,
        ],
    )
    g1, agg2, cnt2 = sc1(ea, src, dst, vc1, b1h, wp2,
                         jnp.zeros((n, d), f32), jnp.zeros((n, 16), f32))

    ea_spec = pl.BlockSpec((be, 1), lambda i: (i, 0))
    g_spec = pl.BlockSpec((be, d), lambda i: (i, 0))
    sums_spec = pl.BlockSpec((8, d), lambda i: (0, 0))
    ea2 = ea.reshape(e, 1)
    sums1 = pl.pallas_call(
        _epass1_l1_body,
        grid=(grid_e,),
        in_specs=[ea_spec, g_spec, _full_spec((2, d)), _full_spec((d, d)),
                  _full_spec((1, d))],
        out_specs=sums_spec,
        out_shape=sds((8, d), f32),
    )(ea2, g1, wp2, l1["A"]["W"], l1["A"]["b"].reshape(1, d))

    node2 = pl.pallas_call(
        _node_stage2_body,
        grid=(1,),
        in_specs=[_full_spec((n, d)), _full_spec((2, n, d)),
                  _full_spec((2, n, 16)), _full_spec((n, d)),
                  _full_spec((1, d)), _full_spec((1, d)),
                  _full_spec((d, d)), _full_spec((1, d)),
                  _full_spec((d, d)), _full_spec((1, d))],
        out_specs=[_full_spec((n, d)), _full_spec((n, d))],
        out_shape=[sds((n, d), f32), sds((n, d), f32)],
    )
    b2h, c2h = node2(
        u1h, agg2, cnt2, h0,
        l1["h_bn_g"].reshape(1, d), l1["h_bn_b"].reshape(1, d),
        l2["B"]["W"], l2["B"]["b"].reshape(1, d),
        l2["C"]["W"], l2["C"]["b"].reshape(1, d))

    sc2 = pl.kernel(
        functools.partial(_sc_layer2_body, ew=ew, nchunk=nchunk),
        out_type=sds((e, d), f32),
        mesh=mesh,
        scratch_types=[
            pltpu.VMEM((_C,), jnp.int32),
            pltpu.VMEM((_C,), jnp.int32),
            pltpu.VMEM((_C, d), f32),
            pltpu.VMEM((_C, d), f32),
            pltpu.SemaphoreType.DMA,
            pltpu.SemaphoreType.DMA,
        ],
    )
    g2 = sc2(src, dst, b2h, c2h)

    bn1 = _row2(l1["e_bn_g"], l1["e_bn_b"])
    bn2 = _row2(l2["e_bn_g"], l2["e_bn_b"])
    sums2 = pl.pallas_call(
        functools.partial(_epass1_l2_body, inv_e=inv_e),
        grid=(grid_e,),
        in_specs=[ea_spec, g_spec, g_spec, _full_spec((2, d)),
                  _full_spec((d, d)), _full_spec((1, d)), _full_spec((8, d)),
                  _full_spec((2, d)), _full_spec((d, d)), _full_spec((1, d))],
        out_specs=sums_spec,
        out_shape=sds((8, d), f32),
    )(ea2, g1, g2, wp2, l1["A"]["W"], l1["A"]["b"].reshape(1, d), sums1,
      bn1, l2["A"]["W"], l2["A"]["b"].reshape(1, d))

    z = pl.pallas_call(
        functools.partial(_final_body, inv_e=inv_e),
        grid=(grid_e,),
        in_specs=[ea_spec, g_spec, g_spec, _full_spec((2, d)),
                  _full_spec((d, d)), _full_spec((1, d)), _full_spec((8, d)),
                  _full_spec((2, d)), _full_spec((d, d)), _full_spec((1, d)),
                  _full_spec((8, d)), _full_spec((2, d)),
                  _full_spec((d, d)), _full_spec((1, d)),
                  _full_spec((d, d)), _full_spec((1, d)),
                  _full_spec((d, 1)), _full_spec((1, 1))],
        out_specs=pl.BlockSpec((be, 1), lambda i: (i, 0)),
        out_shape=sds((e, 1), f32),
    )(ea2, g1, g2, wp2, l1["A"]["W"], l1["A"]["b"].reshape(1, d), sums1,
      bn1, l2["A"]["W"], l2["A"]["b"].reshape(1, d), sums2, bn2,
      mlp[0]["W"], mlp[0]["b"].reshape(1, d),
      mlp[1]["W"], mlp[1]["b"].reshape(1, d),
      mlp[2]["W"], mlp[2]["b"].reshape(1, 1))
    return z


# speedup vs baseline: 1.2197x; 1.2197x over previous
import functools

import jax
import jax.numpy as jnp
from jax import lax
from jax.experimental import pallas as pl
from jax.experimental.pallas import tpu as pltpu
from jax.experimental.pallas import tpu_sc as plsc

_NC = 2
_NS = 16
_NW = _NC * _NS
_C = 80
_EPS = 1e-5
_F32 = jnp.float32


def _relu(v):
    return jnp.maximum(v, 0.0)


def _sigmoid(v):
    return 1.0 / (1.0 + jnp.exp(-v))


def _dot(a, b):
    return jnp.dot(a, b, preferred_element_type=jnp.float32)


def _node_stage1_body(x_ref, wh_ref, bh_ref, u_ref, bu_ref, v_ref, bv_ref,
                      b_ref, bb_ref, c_ref, bc_ref,
                      h0_ref, uh_ref, vc_ref, bh_out_ref):
    x = x_ref[...]
    h0 = _relu(x[:, 0:1] * wh_ref[0:1, :] + x[:, 1:2] * wh_ref[1:2, :]
               + bh_ref[0:1, :])
    h0_ref[...] = h0
    uh_ref[...] = _dot(h0, u_ref[...]) + bu_ref[0:1, :]
    vc_ref[:, 0:128] = _dot(h0, v_ref[...]) + bv_ref[0:1, :]
    vc_ref[:, 128:256] = _dot(h0, c_ref[...]) + bc_ref[0:1, :]
    bh_out_ref[...] = _dot(h0, b_ref[...]) + bb_ref[0:1, :]


def _node_stage2_body(uh_ref, agg2_ref, cntf_ref, h0_ref, g_ref, b_ref,
                      b2_ref, bb2_ref, c2_ref, bc2_ref,
                      b2h_ref, c2h_ref, *, n):
    agg = agg2_ref[0, 0:n, :] + agg2_ref[1, 0:n, :]
    cnt = cntf_ref[0, 0:n, 0:1] + cntf_ref[1, 0:n, 0:1]
    q = uh_ref[...] + agg / jnp.maximum(cnt, 1.0)
    m = jnp.mean(q, axis=0, keepdims=True)
    v = jnp.mean((q - m) ** 2, axis=0, keepdims=True)
    h1 = h0_ref[...] + _relu((q - m) * lax.rsqrt(v + _EPS) * g_ref[0:1, :]
                             + b_ref[0:1, :])
    b2h_ref[...] = _dot(h1, b2_ref[...]) + bb2_ref[0:1, :]
    c2h_ref[...] = _dot(h1, c2_ref[...]) + bc2_ref[0:1, :]


def _e0_block(ea, wp_ref):
    return _relu(ea * wp_ref[0:1, :] + wp_ref[1:2, :])


def _epass1_l1_body(ea_ref, g1_ref, wp_ref, a1_ref, ba1_ref, sums_ref):
    i = pl.program_id(0)
    e0 = _e0_block(ea_ref[...], wp_ref)
    y = _dot(e0, a1_ref[...]) + ba1_ref[0:1, :] + g1_ref[...]

    @pl.when(i == 0)
    def _():
        sums_ref[...] = jnp.zeros_like(sums_ref)

    sums_ref[0:1, :] += jnp.sum(y, axis=0, keepdims=True)
    sums_ref[1:2, :] += jnp.sum(y * y, axis=0, keepdims=True)


def _e1_block(ea, g1, wp_ref, a1_ref, ba1_ref, sums1_ref, bn1_ref, inv_e):
    e0 = _e0_block(ea, wp_ref)
    y1 = _dot(e0, a1_ref[...]) + ba1_ref[0:1, :] + g1
    m1 = sums1_ref[0:1, :] * inv_e
    v1 = sums1_ref[1:2, :] * inv_e - m1 * m1
    return e0 + _relu((y1 - m1) * lax.rsqrt(v1 + _EPS) * bn1_ref[0:1, :]
                      + bn1_ref[1:2, :])


def _epass1_l2_body(ea_ref, g1_ref, g2_ref, wp_ref, a1_ref, ba1_ref,
                    sums1_ref, bn1_ref, a2_ref, ba2_ref, sums2_ref, *,
                    inv_e):
    i = pl.program_id(0)
    e1 = _e1_block(ea_ref[...], g1_ref[...], wp_ref, a1_ref, ba1_ref,
                   sums1_ref, bn1_ref, inv_e)
    y2 = _dot(e1, a2_ref[...]) + ba2_ref[0:1, :] + g2_ref[...]

    @pl.when(i == 0)
    def _():
        sums2_ref[...] = jnp.zeros_like(sums2_ref)

    sums2_ref[0:1, :] += jnp.sum(y2, axis=0, keepdims=True)
    sums2_ref[1:2, :] += jnp.sum(y2 * y2, axis=0, keepdims=True)


def _final_body(ea_ref, g1_ref, g2_ref, wp_ref, a1_ref, ba1_ref, sums1_ref,
                bn1_ref, a2_ref, ba2_ref, sums2_ref, bn2_ref,
                w1_ref, b1_ref, w2_ref, b2_ref, w3_ref, b3_ref, z_ref, *,
                inv_e):
    e1 = _e1_block(ea_ref[...], g1_ref[...], wp_ref, a1_ref, ba1_ref,
                   sums1_ref, bn1_ref, inv_e)
    y2 = _dot(e1, a2_ref[...]) + ba2_ref[0:1, :] + g2_ref[...]
    m2 = sums2_ref[0:1, :] * inv_e
    v2 = sums2_ref[1:2, :] * inv_e - m2 * m2
    e2 = e1 + _relu((y2 - m2) * lax.rsqrt(v2 + _EPS) * bn2_ref[0:1, :]
                    + bn2_ref[1:2, :])
    t = _dot(e2, w1_ref[...]) + b1_ref[0:1, :]
    t = t * _sigmoid(t)
    t = _dot(t, w2_ref[...]) + b2_ref[0:1, :]
    t = t * _sigmoid(t)
    z_ref[...] = _sigmoid(_dot(t, w3_ref[...]) + b3_ref[0:1, :])


def _sc_l1_body(ea_hbm, src_hbm, dst_hbm, vc_hbm, bh_hbm, wp_hbm,
                zero_hbm,
                g_out, agg_out,
                src_v, dst_v, vcrows, brows, msgs, eav, wp_v,
                agg_sh, sem1, sem2, *, n_pad, ew, nchunk):
    c = lax.axis_index("c")
    s = lax.axis_index("s")
    base = (c * _NS + s) * ew
    rows = n_pad // _NS
    r0 = s * rows

    pltpu.sync_copy(wp_hbm, wp_v)
    pltpu.sync_copy(zero_hbm.at[pl.ds(r0, rows)], agg_sh.at[pl.ds(r0, rows)])
    plsc.subcore_barrier()

    def chunk_body(ci, carry):
        b0 = pl.multiple_of(base + ci * _C, 8)
        pltpu.sync_copy(src_hbm.at[pl.ds(b0, _C)], src_v)
        pltpu.sync_copy(dst_hbm.at[pl.ds(b0, _C)], dst_v)
        pltpu.sync_copy(ea_hbm.at[pl.ds(b0, _C)], eav)
        cp1 = pltpu.async_copy(vc_hbm.at[dst_v], vcrows, sem1)
        cp2 = pltpu.async_copy(bh_hbm.at[src_v], brows, sem2)
        cp1.wait()
        cp2.wait()

        def group_body(gi, carry2):
            jb = gi * 16
            av = eav[pl.ds(jb, 16)]
            for jj in range(16):
                a = av[jj]
                j = jb + jj
                for k in range(8):
                    sl = pl.ds(k * 16, 16)
                    e0 = _relu(a * wp_v[0, sl] + wp_v[1, sl])
                    msgs[j, sl] = _sigmoid(e0) * vcrows[j, sl]
                    brows[j, sl] = (brows[j, sl]
                                    + vcrows[j, pl.ds(128 + k * 16, 16)])
            return carry2

        lax.fori_loop(0, _C // 16, group_body, 0)
        pltpu.sync_copy(brows, g_out.at[pl.ds(b0, _C)])
        pltpu.sync_copy(msgs, agg_sh.at[src_v], add=True)
        return carry

    lax.fori_loop(0, nchunk, chunk_body, 0)
    plsc.subcore_barrier()
    pltpu.sync_copy(agg_sh.at[pl.ds(r0, rows)], agg_out.at[c, pl.ds(r0, rows)])


def _sc_cnt_body(src_hbm, zero_hbm, cnt_out,
                 src_v, ones_v, cnt_sh, *, n_pad, ew, nchunk):
    c = lax.axis_index("c")
    s = lax.axis_index("s")
    base = (c * _NS + s) * ew
    rows = n_pad // _NS
    r0 = s * rows

    pltpu.sync_copy(zero_hbm.at[pl.ds(r0, rows)], cnt_sh.at[pl.ds(r0, rows)])
    for j in range(_C):
        for k in range(8):
            ones_v[j, pl.ds(k * 16, 16)] = jnp.full((16,), 1.0, _F32)
    plsc.subcore_barrier()

    def chunk_body(ci, carry):
        b0 = pl.multiple_of(base + ci * _C, 8)
        pltpu.sync_copy(src_hbm.at[pl.ds(b0, _C)], src_v)
        pltpu.sync_copy(ones_v, cnt_sh.at[src_v], add=True)
        return carry

    lax.fori_loop(0, nchunk, chunk_body, 0)
    plsc.subcore_barrier()
    pltpu.sync_copy(cnt_sh.at[pl.ds(r0, rows)], cnt_out.at[c, pl.ds(r0, rows)])


def _sc_l2_body(src_hbm, dst_hbm, b2_hbm, c2_hbm, g_out,
                src_v, dst_v, brows, crows, sem1, sem2, *, ew, nchunk):
    c = lax.axis_index("c")
    s = lax.axis_index("s")
    base = (c * _NS + s) * ew

    def chunk_body(ci, carry):
        b0 = pl.multiple_of(base + ci * _C, 8)
        pltpu.sync_copy(src_hbm.at[pl.ds(b0, _C)], src_v)
        pltpu.sync_copy(dst_hbm.at[pl.ds(b0, _C)], dst_v)
        cp1 = pltpu.async_copy(b2_hbm.at[src_v], brows, sem1)
        cp2 = pltpu.async_copy(c2_hbm.at[dst_v], crows, sem2)
        cp1.wait()
        cp2.wait()

        def edge_body(j, carry2):
            for k in range(8):
                sl = pl.ds(k * 16, 16)
                brows[j, sl] = brows[j, sl] + crows[j, sl]
            return carry2

        lax.fori_loop(0, _C, edge_body, 0)
        pltpu.sync_copy(brows, g_out.at[pl.ds(b0, _C)])
        return carry

    lax.fori_loop(0, nchunk, chunk_body, 0)


def _full_spec(shape):
    return pl.BlockSpec(shape, lambda i: tuple(0 for _ in shape))


def _row2(w, b):
    return jnp.stack([w.reshape(-1), b.reshape(-1)], axis=0)


def kernel(x, edge_attr, edge_index, params):
    n = x.shape[0]
    e = edge_attr.shape[0]
    d = 128
    assert e % (_NW * _C) == 0
    n_pad = -(-n // 128) * 128
    ew = e // _NW
    nchunk = ew // _C
    be = 4000
    grid_e = e // be
    inv_e = 1.0 / e

    src = edge_index[0]
    dst = edge_index[1]
    ea = edge_attr.reshape(e)
    p = params
    l1, l2 = p["layers"][0], p["layers"][1]
    mlp = p["mlp"]

    f32 = jnp.float32
    sds = jax.ShapeDtypeStruct

    node1 = pl.pallas_call(
        _node_stage1_body,
        grid=(1,),
        in_specs=[_full_spec((n, 2))] + [_full_spec(s) for s in
                  [(2, d), (1, d), (d, d), (1, d), (d, d), (1, d),
                   (d, d), (1, d), (d, d), (1, d)]],
        out_specs=[_full_spec((n, d)), _full_spec((n, d)),
                   _full_spec((n, 256)), _full_spec((n, d))],
        out_shape=[sds((n, d), f32), sds((n, d), f32), sds((n, 256), f32),
                   sds((n, d), f32)],
    )
    h0, u1h, vc1, b1h = node1(
        x, p["h_proj"]["W"], p["h_proj"]["b"].reshape(1, d),
        l1["U"]["W"], l1["U"]["b"].reshape(1, d),
        l1["V"]["W"], l1["V"]["b"].reshape(1, d),
        l1["B"]["W"], l1["B"]["b"].reshape(1, d),
        l1["C"]["W"], l1["C"]["b"].reshape(1, d))

    wp2 = _row2(p["e_proj"]["W"], p["e_proj"]["b"])
    mesh = plsc.VectorSubcoreMesh(core_axis_name="c", subcore_axis_name="s")
    sc1 = pl.kernel(
        functools.partial(_sc_l1_body, n_pad=n_pad, ew=ew, nchunk=nchunk),
        out_type=(sds((e, d), f32), sds((2, n_pad, d), f32)),
        mesh=mesh,
        scratch_types=[
            pltpu.VMEM((_C,), jnp.int32),
            pltpu.VMEM((_C,), jnp.int32),
            pltpu.VMEM((_C, 256), f32),
            pltpu.VMEM((_C, d), f32),
            pltpu.VMEM((_C, d), f32),
            pltpu.VMEM((_C,), f32),
            pltpu.VMEM((2, d), f32),
            pltpu.VMEM_SHARED((n_pad, d), f32),
            pltpu.SemaphoreType.DMA,
            pltpu.SemaphoreType.DMA,
        ],
    )
    g1, agg2 = sc1(ea, src, dst, vc1, b1h, wp2, jnp.zeros((n_pad, d), f32))

    sc_cnt = pl.kernel(
        functools.partial(_sc_cnt_body, n_pad=n_pad, ew=ew, nchunk=nchunk),
        out_type=sds((2, n_pad, d), f32),
        mesh=mesh,
        scratch_types=[
            pltpu.VMEM((_C,), jnp.int32),
            pltpu.VMEM((_C, d), f32),
            pltpu.VMEM_SHARED((n_pad, d), f32),
        ],
    )
    cntf = sc_cnt(src, jnp.zeros((n_pad, d), f32))

    ea_spec = pl.BlockSpec((be, 1), lambda i: (i, 0))
    g_spec = pl.BlockSpec((be, d), lambda i: (i, 0))
    sums_spec = pl.BlockSpec((8, d), lambda i: (0, 0))
    ea2 = ea.reshape(e, 1)
    sums1 = pl.pallas_call(
        _epass1_l1_body,
        grid=(grid_e,),
        in_specs=[ea_spec, g_spec, _full_spec((2, d)), _full_spec((d, d)),
                  _full_spec((1, d))],
        out_specs=sums_spec,
        out_shape=sds((8, d), f32),
    )(ea2, g1, wp2, l1["A"]["W"], l1["A"]["b"].reshape(1, d))

    node2 = pl.pallas_call(
        functools.partial(_node_stage2_body, n=n),
        grid=(1,),
        in_specs=[_full_spec((n, d)), _full_spec((2, n_pad, d)),
                  _full_spec((2, n_pad, d)), _full_spec((n, d)),
                  _full_spec((1, d)), _full_spec((1, d)),
                  _full_spec((d, d)), _full_spec((1, d)),
                  _full_spec((d, d)), _full_spec((1, d))],
        out_specs=[_full_spec((n, d)), _full_spec((n, d))],
        out_shape=[sds((n, d), f32), sds((n, d), f32)],
    )
    b2h, c2h = node2(
        u1h, agg2, cntf, h0,
        l1["h_bn_g"].reshape(1, d), l1["h_bn_b"].reshape(1, d),
        l2["B"]["W"], l2["B"]["b"].reshape(1, d),
        l2["C"]["W"], l2["C"]["b"].reshape(1, d))

    sc2 = pl.kernel(
        functools.partial(_sc_l2_body, ew=ew, nchunk=nchunk),
        out_type=sds((e, d), f32),
        mesh=mesh,
        scratch_types=[
            pltpu.VMEM((_C,), jnp.int32),
            pltpu.VMEM((_C,), jnp.int32),
            pltpu.VMEM((_C, d), f32),
            pltpu.VMEM((_C, d), f32),
            pltpu.SemaphoreType.DMA,
            pltpu.SemaphoreType.DMA,
        ],
    )
    g2 = sc2(src, dst, b2h, c2h)

    bn1 = _row2(l1["e_bn_g"], l1["e_bn_b"])
    bn2 = _row2(l2["e_bn_g"], l2["e_bn_b"])
    sums2 = pl.pallas_call(
        functools.partial(_epass1_l2_body, inv_e=inv_e),
        grid=(grid_e,),
        in_specs=[ea_spec, g_spec, g_spec, _full_spec((2, d)),
                  _full_spec((d, d)), _full_spec((1, d)), _full_spec((8, d)),
                  _full_spec((2, d)), _full_spec((d, d)), _full_spec((1, d))],
        out_specs=sums_spec,
        out_shape=sds((8, d), f32),
    )(ea2, g1, g2, wp2, l1["A"]["W"], l1["A"]["b"].reshape(1, d), sums1,
      bn1, l2["A"]["W"], l2["A"]["b"].reshape(1, d))

    z = pl.pallas_call(
        functools.partial(_final_body, inv_e=inv_e),
        grid=(grid_e,),
        in_specs=[ea_spec, g_spec, g_spec, _full_spec((2, d)),
                  _full_spec((d, d)), _full_spec((1, d)), _full_spec((8, d)),
                  _full_spec((2, d)), _full_spec((d, d)), _full_spec((1, d)),
                  _full_spec((8, d)), _full_spec((2, d)),
                  _full_spec((d, d)), _full_spec((1, d)),
                  _full_spec((d, d)), _full_spec((1, d)),
                  _full_spec((d, 1)), _full_spec((1, 1))],
        out_specs=pl.BlockSpec((be, 1), lambda i: (i, 0)),
        out_shape=sds((e, 1), f32),
    )(ea2, g1, g2, wp2, l1["A"]["W"], l1["A"]["b"].reshape(1, d), sums1,
      bn1, l2["A"]["W"], l2["A"]["b"].reshape(1, d), sums2, bn2,
      mlp[0]["W"], mlp[0]["b"].reshape(1, d),
      mlp[1]["W"], mlp[1]["b"].reshape(1, d),
      mlp[2]["W"], mlp[2]["b"].reshape(1, 1))
    return z


# speedup vs baseline: 2.1904x; 1.7959x over previous
import functools

import jax
import jax.numpy as jnp
from jax import lax
from jax.experimental import pallas as pl
from jax.experimental.pallas import tpu as pltpu
from jax.experimental.pallas import tpu_sc as plsc

_NC = 2
_NS = 16
_NW = _NC * _NS
_C = 80
_EPS = 1e-5
_F32 = jnp.float32


def _relu(v):
    return jnp.maximum(v, 0.0)


def _sigmoid(v):
    return 1.0 / (1.0 + jnp.exp(-v))


def _dot(a, b):
    return jnp.dot(a, b, preferred_element_type=jnp.float32)


def _node_stage1_body(x_ref, wh_ref, bh_ref, u_ref, bu_ref, v_ref, bv_ref,
                      b_ref, bb_ref, c_ref, bc_ref,
                      h0_ref, uh_ref, vc_ref, bh_out_ref):
    x = x_ref[...]
    h0 = _relu(x[:, 0:1] * wh_ref[0:1, :] + x[:, 1:2] * wh_ref[1:2, :]
               + bh_ref[0:1, :])
    h0_ref[...] = h0
    uh_ref[...] = _dot(h0, u_ref[...]) + bu_ref[0:1, :]
    vc_ref[:, 0:128] = _dot(h0, v_ref[...]) + bv_ref[0:1, :]
    vc_ref[:, 128:256] = _dot(h0, c_ref[...]) + bc_ref[0:1, :]
    bh_out_ref[...] = _dot(h0, b_ref[...]) + bb_ref[0:1, :]


def _node_stage2_body(uh_ref, agg2_ref, cntf_ref, h0_ref, g_ref, b_ref,
                      b2_ref, bb2_ref, c2_ref, bc2_ref,
                      b2h_ref, c2h_ref, *, n):
    agg = agg2_ref[0, 0:n, :] + agg2_ref[1, 0:n, :]
    cnt = cntf_ref[0, 0:n, 0:1] + cntf_ref[1, 0:n, 0:1]
    q = uh_ref[...] + agg / jnp.maximum(cnt, 1.0)
    m = jnp.mean(q, axis=0, keepdims=True)
    v = jnp.mean((q - m) ** 2, axis=0, keepdims=True)
    h1 = h0_ref[...] + _relu((q - m) * lax.rsqrt(v + _EPS) * g_ref[0:1, :]
                             + b_ref[0:1, :])
    b2h_ref[...] = _dot(h1, b2_ref[...]) + bb2_ref[0:1, :]
    c2h_ref[...] = _dot(h1, c2_ref[...]) + bc2_ref[0:1, :]


def _e0_block(ea, wp_ref):
    return _relu(ea * wp_ref[0:1, :] + wp_ref[1:2, :])


def _sgate_body(ea_ref, wp_ref, s_ref):
    s_ref[...] = _sigmoid(_e0_block(ea_ref[...], wp_ref))


def _epass1_l1_body(ea_ref, g1_ref, wp_ref, a1_ref, ba1_ref, sums_ref):
    i = pl.program_id(0)
    e0 = _e0_block(ea_ref[...], wp_ref)
    y = _dot(e0, a1_ref[...]) + ba1_ref[0:1, :] + g1_ref[...]

    @pl.when(i == 0)
    def _():
        sums_ref[...] = jnp.zeros_like(sums_ref)

    sums_ref[0:1, :] += jnp.sum(y, axis=0, keepdims=True)
    sums_ref[1:2, :] += jnp.sum(y * y, axis=0, keepdims=True)


def _e1_block(ea, g1, wp_ref, a1_ref, ba1_ref, sums1_ref, bn1_ref, inv_e):
    e0 = _e0_block(ea, wp_ref)
    y1 = _dot(e0, a1_ref[...]) + ba1_ref[0:1, :] + g1
    m1 = sums1_ref[0:1, :] * inv_e
    v1 = sums1_ref[1:2, :] * inv_e - m1 * m1
    return e0 + _relu((y1 - m1) * lax.rsqrt(v1 + _EPS) * bn1_ref[0:1, :]
                      + bn1_ref[1:2, :])


def _epass1_l2_body(ea_ref, g1_ref, g2_ref, wp_ref, a1_ref, ba1_ref,
                    sums1_ref, bn1_ref, a2_ref, ba2_ref, sums2_ref, *,
                    inv_e):
    i = pl.program_id(0)
    e1 = _e1_block(ea_ref[...], g1_ref[...], wp_ref, a1_ref, ba1_ref,
                   sums1_ref, bn1_ref, inv_e)
    y2 = _dot(e1, a2_ref[...]) + ba2_ref[0:1, :] + g2_ref[...]

    @pl.when(i == 0)
    def _():
        sums2_ref[...] = jnp.zeros_like(sums2_ref)

    sums2_ref[0:1, :] += jnp.sum(y2, axis=0, keepdims=True)
    sums2_ref[1:2, :] += jnp.sum(y2 * y2, axis=0, keepdims=True)


def _final_body(ea_ref, g1_ref, g2_ref, wp_ref, a1_ref, ba1_ref, sums1_ref,
                bn1_ref, a2_ref, ba2_ref, sums2_ref, bn2_ref,
                w1_ref, b1_ref, w2_ref, b2_ref, w3_ref, b3_ref, z_ref, *,
                inv_e):
    e1 = _e1_block(ea_ref[...], g1_ref[...], wp_ref, a1_ref, ba1_ref,
                   sums1_ref, bn1_ref, inv_e)
    y2 = _dot(e1, a2_ref[...]) + ba2_ref[0:1, :] + g2_ref[...]
    m2 = sums2_ref[0:1, :] * inv_e
    v2 = sums2_ref[1:2, :] * inv_e - m2 * m2
    e2 = e1 + _relu((y2 - m2) * lax.rsqrt(v2 + _EPS) * bn2_ref[0:1, :]
                    + bn2_ref[1:2, :])
    t = _dot(e2, w1_ref[...]) + b1_ref[0:1, :]
    t = t * _sigmoid(t)
    t = _dot(t, w2_ref[...]) + b2_ref[0:1, :]
    t = t * _sigmoid(t)
    z_ref[...] = _sigmoid(_dot(t, w3_ref[...]) + b3_ref[0:1, :])


def _sc_l1_body(s_hbm, src_hbm, dst_hbm, vc_hbm, bh_hbm,
                zero_hbm,
                g_out, agg_out,
                src_v, dst_v, vcrows, brows, msgs,
                agg_sh, sem1, sem2, *, n_pad, ew, nchunk):
    c = lax.axis_index("c")
    s = lax.axis_index("s")
    base = (c * _NS + s) * ew
    rows = n_pad // _NS
    r0 = s * rows

    pltpu.sync_copy(zero_hbm.at[pl.ds(r0, rows)], agg_sh.at[pl.ds(r0, rows)])
    plsc.subcore_barrier()

    def chunk_body(ci, carry):
        b0 = pl.multiple_of(base + ci * _C, 8)
        pltpu.sync_copy(src_hbm.at[pl.ds(b0, _C)], src_v)
        pltpu.sync_copy(dst_hbm.at[pl.ds(b0, _C)], dst_v)
        cp1 = pltpu.async_copy(vc_hbm.at[dst_v], vcrows, sem1)
        cp2 = pltpu.async_copy(bh_hbm.at[src_v], brows, sem2)
        pltpu.sync_copy(s_hbm.at[pl.ds(b0, _C)], msgs)
        cp1.wait()
        cp2.wait()

        def edge_body(j, carry2):
            for k in range(8):
                sl = pl.ds(k * 16, 16)
                msgs[j, sl] = msgs[j, sl] * vcrows[j, sl]
                brows[j, sl] = (brows[j, sl]
                                + vcrows[j, pl.ds(128 + k * 16, 16)])
            return carry2

        lax.fori_loop(0, _C, edge_body, 0)
        pltpu.sync_copy(brows, g_out.at[pl.ds(b0, _C)])
        pltpu.sync_copy(msgs, agg_sh.at[src_v], add=True)
        return carry

    lax.fori_loop(0, nchunk, chunk_body, 0)
    plsc.subcore_barrier()
    pltpu.sync_copy(agg_sh.at[pl.ds(r0, rows)], agg_out.at[c, pl.ds(r0, rows)])


def _sc_cnt_body(src_hbm, zero_hbm, cnt_out,
                 src_v, ones_v, cnt_sh, *, n_pad, ew, nchunk):
    c = lax.axis_index("c")
    s = lax.axis_index("s")
    base = (c * _NS + s) * ew
    rows = n_pad // _NS
    r0 = s * rows

    pltpu.sync_copy(zero_hbm.at[pl.ds(r0, rows)], cnt_sh.at[pl.ds(r0, rows)])
    for j in range(_C):
        for k in range(8):
            ones_v[j, pl.ds(k * 16, 16)] = jnp.full((16,), 1.0, _F32)
    plsc.subcore_barrier()

    def chunk_body(ci, carry):
        b0 = pl.multiple_of(base + ci * _C, 8)
        pltpu.sync_copy(src_hbm.at[pl.ds(b0, _C)], src_v)
        pltpu.sync_copy(ones_v, cnt_sh.at[src_v], add=True)
        return carry

    lax.fori_loop(0, nchunk, chunk_body, 0)
    plsc.subcore_barrier()
    pltpu.sync_copy(cnt_sh.at[pl.ds(r0, rows)], cnt_out.at[c, pl.ds(r0, rows)])


def _sc_l2_body(src_hbm, dst_hbm, b2_hbm, c2_hbm, g_out,
                src_v, dst_v, brows, crows, sem1, sem2, *, ew, nchunk):
    c = lax.axis_index("c")
    s = lax.axis_index("s")
    base = (c * _NS + s) * ew

    def chunk_body(ci, carry):
        b0 = pl.multiple_of(base + ci * _C, 8)
        pltpu.sync_copy(src_hbm.at[pl.ds(b0, _C)], src_v)
        pltpu.sync_copy(dst_hbm.at[pl.ds(b0, _C)], dst_v)
        cp1 = pltpu.async_copy(b2_hbm.at[src_v], brows, sem1)
        cp2 = pltpu.async_copy(c2_hbm.at[dst_v], crows, sem2)
        cp1.wait()
        cp2.wait()

        def edge_body(j, carry2):
            for k in range(8):
                sl = pl.ds(k * 16, 16)
                brows[j, sl] = brows[j, sl] + crows[j, sl]
            return carry2

        lax.fori_loop(0, _C, edge_body, 0)
        pltpu.sync_copy(brows, g_out.at[pl.ds(b0, _C)])
        return carry

    lax.fori_loop(0, nchunk, chunk_body, 0)


def _full_spec(shape):
    return pl.BlockSpec(shape, lambda i: tuple(0 for _ in shape))


def _row2(w, b):
    return jnp.stack([w.reshape(-1), b.reshape(-1)], axis=0)


def kernel(x, edge_attr, edge_index, params):
    n = x.shape[0]
    e = edge_attr.shape[0]
    d = 128
    assert e % (_NW * _C) == 0
    n_pad = -(-n // 128) * 128
    ew = e // _NW
    nchunk = ew // _C
    be = 4000
    grid_e = e // be
    inv_e = 1.0 / e

    src = edge_index[0]
    dst = edge_index[1]
    ea = edge_attr.reshape(e)
    p = params
    l1, l2 = p["layers"][0], p["layers"][1]
    mlp = p["mlp"]

    f32 = jnp.float32
    sds = jax.ShapeDtypeStruct

    node1 = pl.pallas_call(
        _node_stage1_body,
        grid=(1,),
        in_specs=[_full_spec((n, 2))] + [_full_spec(s) for s in
                  [(2, d), (1, d), (d, d), (1, d), (d, d), (1, d),
                   (d, d), (1, d), (d, d), (1, d)]],
        out_specs=[_full_spec((n, d)), _full_spec((n, d)),
                   _full_spec((n, 256)), _full_spec((n, d))],
        out_shape=[sds((n, d), f32), sds((n, d), f32), sds((n, 256), f32),
                   sds((n, d), f32)],
    )
    h0, u1h, vc1, b1h = node1(
        x, p["h_proj"]["W"], p["h_proj"]["b"].reshape(1, d),
        l1["U"]["W"], l1["U"]["b"].reshape(1, d),
        l1["V"]["W"], l1["V"]["b"].reshape(1, d),
        l1["B"]["W"], l1["B"]["b"].reshape(1, d),
        l1["C"]["W"], l1["C"]["b"].reshape(1, d))

    wp2 = _row2(p["e_proj"]["W"], p["e_proj"]["b"])
    be_s = 8000
    ea2 = ea.reshape(e, 1)
    sgate = pl.pallas_call(
        _sgate_body,
        grid=(e // be_s,),
        in_specs=[pl.BlockSpec((be_s, 1), lambda i: (i, 0)),
                  _full_spec((2, d))],
        out_specs=pl.BlockSpec((be_s, d), lambda i: (i, 0)),
        out_shape=sds((e, d), f32),
    )
    sgv = sgate(ea2, wp2)

    mesh = plsc.VectorSubcoreMesh(core_axis_name="c", subcore_axis_name="s")
    sc1 = pl.kernel(
        functools.partial(_sc_l1_body, n_pad=n_pad, ew=ew, nchunk=nchunk),
        out_type=(sds((e, d), f32), sds((2, n_pad, d), f32)),
        mesh=mesh,
        scratch_types=[
            pltpu.VMEM((_C,), jnp.int32),
            pltpu.VMEM((_C,), jnp.int32),
            pltpu.VMEM((_C, 256), f32),
            pltpu.VMEM((_C, d), f32),
            pltpu.VMEM((_C, d), f32),
            pltpu.VMEM_SHARED((n_pad, d), f32),
            pltpu.SemaphoreType.DMA,
            pltpu.SemaphoreType.DMA,
        ],
    )
    g1, agg2 = sc1(sgv, src, dst, vc1, b1h, jnp.zeros((n_pad, d), f32))

    sc_cnt = pl.kernel(
        functools.partial(_sc_cnt_body, n_pad=n_pad, ew=ew, nchunk=nchunk),
        out_type=sds((2, n_pad, d), f32),
        mesh=mesh,
        scratch_types=[
            pltpu.VMEM((_C,), jnp.int32),
            pltpu.VMEM((_C, d), f32),
            pltpu.VMEM_SHARED((n_pad, d), f32),
        ],
    )
    cntf = sc_cnt(src, jnp.zeros((n_pad, d), f32))

    ea_spec = pl.BlockSpec((be, 1), lambda i: (i, 0))
    g_spec = pl.BlockSpec((be, d), lambda i: (i, 0))
    sums_spec = pl.BlockSpec((8, d), lambda i: (0, 0))
    sums1 = pl.pallas_call(
        _epass1_l1_body,
        grid=(grid_e,),
        in_specs=[ea_spec, g_spec, _full_spec((2, d)), _full_spec((d, d)),
                  _full_spec((1, d))],
        out_specs=sums_spec,
        out_shape=sds((8, d), f32),
    )(ea2, g1, wp2, l1["A"]["W"], l1["A"]["b"].reshape(1, d))

    node2 = pl.pallas_call(
        functools.partial(_node_stage2_body, n=n),
        grid=(1,),
        in_specs=[_full_spec((n, d)), _full_spec((2, n_pad, d)),
                  _full_spec((2, n_pad, d)), _full_spec((n, d)),
                  _full_spec((1, d)), _full_spec((1, d)),
                  _full_spec((d, d)), _full_spec((1, d)),
                  _full_spec((d, d)), _full_spec((1, d))],
        out_specs=[_full_spec((n, d)), _full_spec((n, d))],
        out_shape=[sds((n, d), f32), sds((n, d), f32)],
    )
    b2h, c2h = node2(
        u1h, agg2, cntf, h0,
        l1["h_bn_g"].reshape(1, d), l1["h_bn_b"].reshape(1, d),
        l2["B"]["W"], l2["B"]["b"].reshape(1, d),
        l2["C"]["W"], l2["C"]["b"].reshape(1, d))

    sc2 = pl.kernel(
        functools.partial(_sc_l2_body, ew=ew, nchunk=nchunk),
        out_type=sds((e, d), f32),
        mesh=mesh,
        scratch_types=[
            pltpu.VMEM((_C,), jnp.int32),
            pltpu.VMEM((_C,), jnp.int32),
            pltpu.VMEM((_C, d), f32),
            pltpu.VMEM((_C, d), f32),
            pltpu.SemaphoreType.DMA,
            pltpu.SemaphoreType.DMA,
        ],
    )
    g2 = sc2(src, dst, b2h, c2h)

    bn1 = _row2(l1["e_bn_g"], l1["e_bn_b"])
    bn2 = _row2(l2["e_bn_g"], l2["e_bn_b"])
    sums2 = pl.pallas_call(
        functools.partial(_epass1_l2_body, inv_e=inv_e),
        grid=(grid_e,),
        in_specs=[ea_spec, g_spec, g_spec, _full_spec((2, d)),
                  _full_spec((d, d)), _full_spec((1, d)), _full_spec((8, d)),
                  _full_spec((2, d)), _full_spec((d, d)), _full_spec((1, d))],
        out_specs=sums_spec,
        out_shape=sds((8, d), f32),
    )(ea2, g1, g2, wp2, l1["A"]["W"], l1["A"]["b"].reshape(1, d), sums1,
      bn1, l2["A"]["W"], l2["A"]["b"].reshape(1, d))

    z = pl.pallas_call(
        functools.partial(_final_body, inv_e=inv_e),
        grid=(grid_e,),
        in_specs=[ea_spec, g_spec, g_spec, _full_spec((2, d)),
                  _full_spec((d, d)), _full_spec((1, d)), _full_spec((8, d)),
                  _full_spec((2, d)), _full_spec((d, d)), _full_spec((1, d)),
                  _full_spec((8, d)), _full_spec((2, d)),
                  _full_spec((d, d)), _full_spec((1, d)),
                  _full_spec((d, d)), _full_spec((1, d)),
                  _full_spec((d, 1)), _full_spec((1, 1))],
        out_specs=pl.BlockSpec((be, 1), lambda i: (i, 0)),
        out_shape=sds((e, 1), f32),
    )(ea2, g1, g2, wp2, l1["A"]["W"], l1["A"]["b"].reshape(1, d), sums1,
      bn1, l2["A"]["W"], l2["A"]["b"].reshape(1, d), sums2, bn2,
      mlp[0]["W"], mlp[0]["b"].reshape(1, d),
      mlp[1]["W"], mlp[1]["b"].reshape(1, d),
      mlp[2]["W"], mlp[2]["b"].reshape(1, 1))
    return z


# speedup vs baseline: 2.2406x; 1.0229x over previous
import functools

import jax
import jax.numpy as jnp
from jax import lax
from jax.experimental import pallas as pl
from jax.experimental.pallas import tpu as pltpu
from jax.experimental.pallas import tpu_sc as plsc

_NC = 2
_NS = 16
_NW = _NC * _NS
_C = 128
_EPS = 1e-5
_F32 = jnp.float32


def _relu(v):
    return jnp.maximum(v, 0.0)


def _sigmoid(v):
    return 1.0 / (1.0 + jnp.exp(-v))


def _dot(a, b):
    return jnp.dot(a, b, preferred_element_type=jnp.float32)


def _node_stage1_body(x_ref, wh_ref, bh_ref, u_ref, bu_ref, v_ref, bv_ref,
                      b_ref, bb_ref, c_ref, bc_ref,
                      h0_ref, uh_ref, vc_ref, bh_out_ref):
    x = x_ref[...]
    h0 = _relu(x[:, 0:1] * wh_ref[0:1, :] + x[:, 1:2] * wh_ref[1:2, :]
               + bh_ref[0:1, :])
    h0_ref[...] = h0
    uh_ref[...] = _dot(h0, u_ref[...]) + bu_ref[0:1, :]
    vc_ref[:, 0:128] = _dot(h0, v_ref[...]) + bv_ref[0:1, :]
    vc_ref[:, 128:256] = _dot(h0, c_ref[...]) + bc_ref[0:1, :]
    bh_out_ref[...] = _dot(h0, b_ref[...]) + bb_ref[0:1, :]


def _node_stage2_body(uh_ref, agg2_ref, cntf_ref, h0_ref, g_ref, b_ref,
                      b2_ref, bb2_ref, c2_ref, bc2_ref,
                      b2h_ref, c2h_ref, *, n):
    agg = agg2_ref[0, 0:n, :] + agg2_ref[1, 0:n, :]
    cnt = cntf_ref[0, 0:n, 0:1] + cntf_ref[1, 0:n, 0:1]
    q = uh_ref[...] + agg / jnp.maximum(cnt, 1.0)
    m = jnp.mean(q, axis=0, keepdims=True)
    v = jnp.mean((q - m) ** 2, axis=0, keepdims=True)
    h1 = h0_ref[...] + _relu((q - m) * lax.rsqrt(v + _EPS) * g_ref[0:1, :]
                             + b_ref[0:1, :])
    b2h_ref[...] = _dot(h1, b2_ref[...]) + bb2_ref[0:1, :]
    c2h_ref[...] = _dot(h1, c2_ref[...]) + bc2_ref[0:1, :]


def _e0_block(ea, wp_ref):
    return _relu(ea * wp_ref[0:1, :] + wp_ref[1:2, :])


def _sgate_body(ea_ref, wp_ref, s_ref):
    s_ref[...] = _sigmoid(_e0_block(ea_ref[...], wp_ref))


def _epass1_l1_body(ea_ref, g1_ref, wp_ref, a1_ref, ba1_ref, sums_ref):
    i = pl.program_id(0)
    e0 = _e0_block(ea_ref[...], wp_ref)
    y = _dot(e0, a1_ref[...]) + ba1_ref[0:1, :] + g1_ref[...]

    @pl.when(i == 0)
    def _():
        sums_ref[...] = jnp.zeros_like(sums_ref)

    sums_ref[0:1, :] += jnp.sum(y, axis=0, keepdims=True)
    sums_ref[1:2, :] += jnp.sum(y * y, axis=0, keepdims=True)


def _e1_block(ea, g1, wp_ref, a1_ref, ba1_ref, sums1_ref, bn1_ref, inv_e):
    e0 = _e0_block(ea, wp_ref)
    y1 = _dot(e0, a1_ref[...]) + ba1_ref[0:1, :] + g1
    m1 = sums1_ref[0:1, :] * inv_e
    v1 = sums1_ref[1:2, :] * inv_e - m1 * m1
    return e0 + _relu((y1 - m1) * lax.rsqrt(v1 + _EPS) * bn1_ref[0:1, :]
                      + bn1_ref[1:2, :])


def _epass1_l2_body(ea_ref, g1_ref, g2_ref, wp_ref, a1_ref, ba1_ref,
                    sums1_ref, bn1_ref, a2_ref, ba2_ref, sums2_ref, *,
                    inv_e):
    i = pl.program_id(0)
    e1 = _e1_block(ea_ref[...], g1_ref[...], wp_ref, a1_ref, ba1_ref,
                   sums1_ref, bn1_ref, inv_e)
    y2 = _dot(e1, a2_ref[...]) + ba2_ref[0:1, :] + g2_ref[...]

    @pl.when(i == 0)
    def _():
        sums2_ref[...] = jnp.zeros_like(sums2_ref)

    sums2_ref[0:1, :] += jnp.sum(y2, axis=0, keepdims=True)
    sums2_ref[1:2, :] += jnp.sum(y2 * y2, axis=0, keepdims=True)


def _final_body(ea_ref, g1_ref, g2_ref, wp_ref, a1_ref, ba1_ref, sums1_ref,
                bn1_ref, a2_ref, ba2_ref, sums2_ref, bn2_ref,
                w1_ref, b1_ref, w2_ref, b2_ref, w3_ref, b3_ref, z_ref, *,
                inv_e):
    e1 = _e1_block(ea_ref[...], g1_ref[...], wp_ref, a1_ref, ba1_ref,
                   sums1_ref, bn1_ref, inv_e)
    y2 = _dot(e1, a2_ref[...]) + ba2_ref[0:1, :] + g2_ref[...]
    m2 = sums2_ref[0:1, :] * inv_e
    v2 = sums2_ref[1:2, :] * inv_e - m2 * m2
    e2 = e1 + _relu((y2 - m2) * lax.rsqrt(v2 + _EPS) * bn2_ref[0:1, :]
                    + bn2_ref[1:2, :])
    t = _dot(e2, w1_ref[...]) + b1_ref[0:1, :]
    t = t * _sigmoid(t)
    t = _dot(t, w2_ref[...]) + b2_ref[0:1, :]
    t = t * _sigmoid(t)
    z_ref[...] = _sigmoid(_dot(t, w3_ref[...]) + b3_ref[0:1, :])


def _sc_l1_body(s_hbm, src_hbm, dst_hbm, vc_hbm, bh_hbm,
                zero_hbm,
                g_out, agg_out,
                src_v, dst_v, vcrows, brows, msgs,
                agg_sh, sem1, sem2, *, n_pad, nbase, nrem, cc):
    c = lax.axis_index("c")
    s = lax.axis_index("s")
    w = c * _NS + s
    nw = nbase + jnp.where(w < nrem, 1, 0)
    rows = n_pad // _NS
    r0 = s * rows

    pltpu.sync_copy(zero_hbm.at[pl.ds(r0, rows)], agg_sh.at[pl.ds(r0, rows)])
    plsc.subcore_barrier()

    def chunk_body(ci, carry):
        b0 = pl.multiple_of((ci * _NW + w) * cc, 8)
        pltpu.sync_copy(src_hbm.at[pl.ds(b0, cc)], src_v)
        pltpu.sync_copy(dst_hbm.at[pl.ds(b0, cc)], dst_v)
        cp1 = pltpu.async_copy(vc_hbm.at[dst_v], vcrows, sem1)
        cp2 = pltpu.async_copy(bh_hbm.at[src_v], brows, sem2)
        pltpu.sync_copy(s_hbm.at[pl.ds(b0, cc)], msgs)
        cp1.wait()
        cp2.wait()

        def edge_body(j, carry2):
            for k in range(8):
                sl = pl.ds(k * 16, 16)
                msgs[j, sl] = msgs[j, sl] * vcrows[j, sl]
                brows[j, sl] = (brows[j, sl]
                                + vcrows[j, pl.ds(128 + k * 16, 16)])
            return carry2

        lax.fori_loop(0, cc, edge_body, 0)
        pltpu.sync_copy(brows, g_out.at[pl.ds(b0, cc)])
        pltpu.sync_copy(msgs, agg_sh.at[src_v], add=True)
        return carry

    lax.fori_loop(0, nw, chunk_body, 0)
    plsc.subcore_barrier()
    pltpu.sync_copy(agg_sh.at[pl.ds(r0, rows)], agg_out.at[c, pl.ds(r0, rows)])


def _sc_cnt_body(src_hbm, zero_hbm, cnt_out,
                 src_v, ones_v, cnt_sh, *, n_pad, nbase, nrem, cc):
    c = lax.axis_index("c")
    s = lax.axis_index("s")
    w = c * _NS + s
    nw = nbase + jnp.where(w < nrem, 1, 0)
    rows = n_pad // _NS
    r0 = s * rows

    pltpu.sync_copy(zero_hbm.at[pl.ds(r0, rows)], cnt_sh.at[pl.ds(r0, rows)])
    for j in range(cc):
        for k in range(8):
            ones_v[j, pl.ds(k * 16, 16)] = jnp.full((16,), 1.0, _F32)
    plsc.subcore_barrier()

    def chunk_body(ci, carry):
        b0 = pl.multiple_of((ci * _NW + w) * cc, 8)
        pltpu.sync_copy(src_hbm.at[pl.ds(b0, cc)], src_v)
        pltpu.sync_copy(ones_v, cnt_sh.at[src_v], add=True)
        return carry

    lax.fori_loop(0, nw, chunk_body, 0)
    plsc.subcore_barrier()
    pltpu.sync_copy(cnt_sh.at[pl.ds(r0, rows)], cnt_out.at[c, pl.ds(r0, rows)])


def _sc_l2_body(src_hbm, dst_hbm, b2_hbm, c2_hbm, g_out,
                src_v, dst_v, brows, crows, sem1, sem2, *, nbase, nrem, cc):
    c = lax.axis_index("c")
    s = lax.axis_index("s")
    w = c * _NS + s
    nw = nbase + jnp.where(w < nrem, 1, 0)

    def chunk_body(ci, carry):
        b0 = pl.multiple_of((ci * _NW + w) * cc, 8)
        pltpu.sync_copy(src_hbm.at[pl.ds(b0, cc)], src_v)
        pltpu.sync_copy(dst_hbm.at[pl.ds(b0, cc)], dst_v)
        cp1 = pltpu.async_copy(b2_hbm.at[src_v], brows, sem1)
        cp2 = pltpu.async_copy(c2_hbm.at[dst_v], crows, sem2)
        cp1.wait()
        cp2.wait()

        def edge_body(j, carry2):
            for k in range(8):
                sl = pl.ds(k * 16, 16)
                brows[j, sl] = brows[j, sl] + crows[j, sl]
            return carry2

        lax.fori_loop(0, cc, edge_body, 0)
        pltpu.sync_copy(brows, g_out.at[pl.ds(b0, cc)])
        return carry

    lax.fori_loop(0, nw, chunk_body, 0)


def _full_spec(shape):
    return pl.BlockSpec(shape, lambda i: tuple(0 for _ in shape))


def _row2(w, b):
    return jnp.stack([w.reshape(-1), b.reshape(-1)], axis=0)


def kernel(x, edge_attr, edge_index, params):
    n = x.shape[0]
    e = edge_attr.shape[0]
    d = 128
    ca, cb = 80, 128
    assert e % ca == 0 and e % cb == 0
    n_pad = -(-n // 128) * 128
    nbase_a, nrem_a = (e // ca) // _NW, (e // ca) % _NW
    nbase_b, nrem_b = (e // cb) // _NW, (e // cb) % _NW
    be = 4000
    grid_e = e // be
    inv_e = 1.0 / e

    src = edge_index[0]
    dst = edge_index[1]
    ea = edge_attr.reshape(e)
    p = params
    l1, l2 = p["layers"][0], p["layers"][1]
    mlp = p["mlp"]

    f32 = jnp.float32
    sds = jax.ShapeDtypeStruct

    node1 = pl.pallas_call(
        _node_stage1_body,
        grid=(1,),
        in_specs=[_full_spec((n, 2))] + [_full_spec(s) for s in
                  [(2, d), (1, d), (d, d), (1, d), (d, d), (1, d),
                   (d, d), (1, d), (d, d), (1, d)]],
        out_specs=[_full_spec((n, d)), _full_spec((n, d)),
                   _full_spec((n, 256)), _full_spec((n, d))],
        out_shape=[sds((n, d), f32), sds((n, d), f32), sds((n, 256), f32),
                   sds((n, d), f32)],
    )
    h0, u1h, vc1, b1h = node1(
        x, p["h_proj"]["W"], p["h_proj"]["b"].reshape(1, d),
        l1["U"]["W"], l1["U"]["b"].reshape(1, d),
        l1["V"]["W"], l1["V"]["b"].reshape(1, d),
        l1["B"]["W"], l1["B"]["b"].reshape(1, d),
        l1["C"]["W"], l1["C"]["b"].reshape(1, d))

    wp2 = _row2(p["e_proj"]["W"], p["e_proj"]["b"])
    be_s = 8000
    ea2 = ea.reshape(e, 1)
    sgate = pl.pallas_call(
        _sgate_body,
        grid=(e // be_s,),
        in_specs=[pl.BlockSpec((be_s, 1), lambda i: (i, 0)),
                  _full_spec((2, d))],
        out_specs=pl.BlockSpec((be_s, d), lambda i: (i, 0)),
        out_shape=sds((e, d), f32),
    )
    sgv = sgate(ea2, wp2)

    mesh = plsc.VectorSubcoreMesh(core_axis_name="c", subcore_axis_name="s")
    sc1 = pl.kernel(
        functools.partial(_sc_l1_body, n_pad=n_pad, nbase=nbase_a, nrem=nrem_a, cc=ca),
        out_type=(sds((e, d), f32), sds((2, n_pad, d), f32)),
        mesh=mesh,
        scratch_types=[
            pltpu.VMEM((ca,), jnp.int32),
            pltpu.VMEM((ca,), jnp.int32),
            pltpu.VMEM((ca, 256), f32),
            pltpu.VMEM((ca, d), f32),
            pltpu.VMEM((ca, d), f32),
            pltpu.VMEM_SHARED((n_pad, d), f32),
            pltpu.SemaphoreType.DMA,
            pltpu.SemaphoreType.DMA,
        ],
    )
    g1, agg2 = sc1(sgv, src, dst, vc1, b1h, jnp.zeros((n_pad, d), f32))

    sc_cnt = pl.kernel(
        functools.partial(_sc_cnt_body, n_pad=n_pad, nbase=nbase_b, nrem=nrem_b, cc=cb),
        out_type=sds((2, n_pad, d), f32),
        mesh=mesh,
        scratch_types=[
            pltpu.VMEM((cb,), jnp.int32),
            pltpu.VMEM((cb, d), f32),
            pltpu.VMEM_SHARED((n_pad, d), f32),
        ],
    )
    cntf = sc_cnt(src, jnp.zeros((n_pad, d), f32))

    ea_spec = pl.BlockSpec((be, 1), lambda i: (i, 0))
    g_spec = pl.BlockSpec((be, d), lambda i: (i, 0))
    sums_spec = pl.BlockSpec((8, d), lambda i: (0, 0))
    sums1 = pl.pallas_call(
        _epass1_l1_body,
        grid=(grid_e,),
        in_specs=[ea_spec, g_spec, _full_spec((2, d)), _full_spec((d, d)),
                  _full_spec((1, d))],
        out_specs=sums_spec,
        out_shape=sds((8, d), f32),
    )(ea2, g1, wp2, l1["A"]["W"], l1["A"]["b"].reshape(1, d))

    node2 = pl.pallas_call(
        functools.partial(_node_stage2_body, n=n),
        grid=(1,),
        in_specs=[_full_spec((n, d)), _full_spec((2, n_pad, d)),
                  _full_spec((2, n_pad, d)), _full_spec((n, d)),
                  _full_spec((1, d)), _full_spec((1, d)),
                  _full_spec((d, d)), _full_spec((1, d)),
                  _full_spec((d, d)), _full_spec((1, d))],
        out_specs=[_full_spec((n, d)), _full_spec((n, d))],
        out_shape=[sds((n, d), f32), sds((n, d), f32)],
    )
    b2h, c2h = node2(
        u1h, agg2, cntf, h0,
        l1["h_bn_g"].reshape(1, d), l1["h_bn_b"].reshape(1, d),
        l2["B"]["W"], l2["B"]["b"].reshape(1, d),
        l2["C"]["W"], l2["C"]["b"].reshape(1, d))

    sc2 = pl.kernel(
        functools.partial(_sc_l2_body, nbase=nbase_b, nrem=nrem_b, cc=cb),
        out_type=sds((e, d), f32),
        mesh=mesh,
        scratch_types=[
            pltpu.VMEM((cb,), jnp.int32),
            pltpu.VMEM((cb,), jnp.int32),
            pltpu.VMEM((cb, d), f32),
            pltpu.VMEM((cb, d), f32),
            pltpu.SemaphoreType.DMA,
            pltpu.SemaphoreType.DMA,
        ],
    )
    g2 = sc2(src, dst, b2h, c2h)

    bn1 = _row2(l1["e_bn_g"], l1["e_bn_b"])
    bn2 = _row2(l2["e_bn_g"], l2["e_bn_b"])
    sums2 = pl.pallas_call(
        functools.partial(_epass1_l2_body, inv_e=inv_e),
        grid=(grid_e,),
        in_specs=[ea_spec, g_spec, g_spec, _full_spec((2, d)),
                  _full_spec((d, d)), _full_spec((1, d)), _full_spec((8, d)),
                  _full_spec((2, d)), _full_spec((d, d)), _full_spec((1, d))],
        out_specs=sums_spec,
        out_shape=sds((8, d), f32),
    )(ea2, g1, g2, wp2, l1["A"]["W"], l1["A"]["b"].reshape(1, d), sums1,
      bn1, l2["A"]["W"], l2["A"]["b"].reshape(1, d))

    z = pl.pallas_call(
        functools.partial(_final_body, inv_e=inv_e),
        grid=(grid_e,),
        in_specs=[ea_spec, g_spec, g_spec, _full_spec((2, d)),
                  _full_spec((d, d)), _full_spec((1, d)), _full_spec((8, d)),
                  _full_spec((2, d)), _full_spec((d, d)), _full_spec((1, d)),
                  _full_spec((8, d)), _full_spec((2, d)),
                  _full_spec((d, d)), _full_spec((1, d)),
                  _full_spec((d, d)), _full_spec((1, d)),
                  _full_spec((d, 1)), _full_spec((1, 1))],
        out_specs=pl.BlockSpec((be, 1), lambda i: (i, 0)),
        out_shape=sds((e, 1), f32),
    )(ea2, g1, g2, wp2, l1["A"]["W"], l1["A"]["b"].reshape(1, d), sums1,
      bn1, l2["A"]["W"], l2["A"]["b"].reshape(1, d), sums2, bn2,
      mlp[0]["W"], mlp[0]["b"].reshape(1, d),
      mlp[1]["W"], mlp[1]["b"].reshape(1, d),
      mlp[2]["W"], mlp[2]["b"].reshape(1, 1))
    return z


# speedup vs baseline: 2.2903x; 1.0222x over previous
import functools

import jax
import jax.numpy as jnp
from jax import lax
from jax.experimental import pallas as pl
from jax.experimental.pallas import tpu as pltpu
from jax.experimental.pallas import tpu_sc as plsc

_NC = 2
_NS = 16
_NW = _NC * _NS
_C = 128
_EPS = 1e-5
_F32 = jnp.float32


def _relu(v):
    return jnp.maximum(v, 0.0)


def _sigmoid(v):
    return 1.0 / (1.0 + jnp.exp(-v))


def _dot(a, b):
    return jnp.dot(a, b, preferred_element_type=jnp.float32)


def _node_stage1_body(x_ref, wh_ref, bh_ref, u_ref, bu_ref, v_ref, bv_ref,
                      b_ref, bb_ref, c_ref, bc_ref,
                      h0_ref, uh_ref, vc_ref, bh_out_ref):
    x = x_ref[...]
    h0 = _relu(x[:, 0:1] * wh_ref[0:1, :] + x[:, 1:2] * wh_ref[1:2, :]
               + bh_ref[0:1, :])
    h0_ref[...] = h0
    uh_ref[...] = _dot(h0, u_ref[...]) + bu_ref[0:1, :]
    vc_ref[:, 0:128] = _dot(h0, v_ref[...]) + bv_ref[0:1, :]
    vc_ref[:, 128:256] = _dot(h0, c_ref[...]) + bc_ref[0:1, :]
    bh_out_ref[...] = _dot(h0, b_ref[...]) + bb_ref[0:1, :]


def _node_stage2_body(uh_ref, agg2_ref, cntf_ref, h0_ref, g_ref, b_ref,
                      b2_ref, bb2_ref, c2_ref, bc2_ref,
                      b2h_ref, c2h_ref, *, n):
    agg = agg2_ref[0, 0:n, :] + agg2_ref[1, 0:n, :]
    cnt = cntf_ref[0, 0:n, 0:1] + cntf_ref[1, 0:n, 0:1]
    q = uh_ref[...] + agg / jnp.maximum(cnt, 1.0)
    m = jnp.mean(q, axis=0, keepdims=True)
    v = jnp.mean((q - m) ** 2, axis=0, keepdims=True)
    h1 = h0_ref[...] + _relu((q - m) * lax.rsqrt(v + _EPS) * g_ref[0:1, :]
                             + b_ref[0:1, :])
    b2h_ref[...] = _dot(h1, b2_ref[...]) + bb2_ref[0:1, :]
    c2h_ref[...] = _dot(h1, c2_ref[...]) + bc2_ref[0:1, :]


def _e0_block(ea, wp_ref):
    return _relu(ea * wp_ref[0:1, :] + wp_ref[1:2, :])


def _sgate_body(ea_ref, wp_ref, s_ref):
    s_ref[...] = _sigmoid(_e0_block(ea_ref[...], wp_ref))


def _epass1_l1_body(ea_ref, g1_ref, wp_ref, a1_ref, ba1_ref, sums_ref):
    i = pl.program_id(0)
    e0 = _e0_block(ea_ref[...], wp_ref)
    y = _dot(e0, a1_ref[...]) + ba1_ref[0:1, :] + g1_ref[...]

    @pl.when(i == 0)
    def _():
        sums_ref[...] = jnp.zeros_like(sums_ref)

    sums_ref[0:1, :] += jnp.sum(y, axis=0, keepdims=True)
    sums_ref[1:2, :] += jnp.sum(y * y, axis=0, keepdims=True)


def _e1_block(ea, g1, wp_ref, a1_ref, ba1_ref, sums1_ref, bn1_ref, inv_e):
    e0 = _e0_block(ea, wp_ref)
    y1 = _dot(e0, a1_ref[...]) + ba1_ref[0:1, :] + g1
    m1 = sums1_ref[0:1, :] * inv_e
    v1 = sums1_ref[1:2, :] * inv_e - m1 * m1
    return e0 + _relu((y1 - m1) * lax.rsqrt(v1 + _EPS) * bn1_ref[0:1, :]
                      + bn1_ref[1:2, :])


def _epass1_l2_body(ea_ref, g1_ref, g2_ref, wp_ref, a1_ref, ba1_ref,
                    sums1_ref, bn1_ref, a2_ref, ba2_ref, sums2_ref, *,
                    inv_e):
    i = pl.program_id(0)
    e1 = _e1_block(ea_ref[...], g1_ref[...], wp_ref, a1_ref, ba1_ref,
                   sums1_ref, bn1_ref, inv_e)
    y2 = _dot(e1, a2_ref[...]) + ba2_ref[0:1, :] + g2_ref[...]

    @pl.when(i == 0)
    def _():
        sums2_ref[...] = jnp.zeros_like(sums2_ref)

    sums2_ref[0:1, :] += jnp.sum(y2, axis=0, keepdims=True)
    sums2_ref[1:2, :] += jnp.sum(y2 * y2, axis=0, keepdims=True)


def _final_body(ea_ref, g1_ref, g2_ref, wp_ref, a1_ref, ba1_ref, sums1_ref,
                bn1_ref, a2_ref, ba2_ref, sums2_ref, bn2_ref,
                w1_ref, b1_ref, w2_ref, b2_ref, w3_ref, b3_ref, z_ref, *,
                inv_e):
    e1 = _e1_block(ea_ref[...], g1_ref[...], wp_ref, a1_ref, ba1_ref,
                   sums1_ref, bn1_ref, inv_e)
    y2 = _dot(e1, a2_ref[...]) + ba2_ref[0:1, :] + g2_ref[...]
    m2 = sums2_ref[0:1, :] * inv_e
    v2 = sums2_ref[1:2, :] * inv_e - m2 * m2
    e2 = e1 + _relu((y2 - m2) * lax.rsqrt(v2 + _EPS) * bn2_ref[0:1, :]
                    + bn2_ref[1:2, :])
    t = _dot(e2, w1_ref[...]) + b1_ref[0:1, :]
    t = t * _sigmoid(t)
    t = _dot(t, w2_ref[...]) + b2_ref[0:1, :]
    t = t * _sigmoid(t)
    z_ref[...] = _sigmoid(_dot(t, w3_ref[...]) + b3_ref[0:1, :])


def _sc_l1_body(s_hbm, src_hbm, dst_hbm, vc_hbm, bh_hbm,
                zero_hbm,
                g_out, agg_out,
                src_v, dst_v, vcrows, brows, msgs,
                agg_sh, sem1, sem2, sem3, *, n_pad, nbase, nrem, cc):
    c = lax.axis_index("c")
    s = lax.axis_index("s")
    w = c * _NS + s
    nw = nbase + jnp.where(w < nrem, 1, 0)
    rows = n_pad // _NS
    r0 = s * rows

    pltpu.sync_copy(zero_hbm.at[pl.ds(r0, rows)], agg_sh.at[pl.ds(r0, rows)])
    plsc.subcore_barrier()

    def chunk_body(ci, carry):
        b0 = pl.multiple_of((ci * _NW + w) * cc, 8)
        cps = pltpu.async_copy(s_hbm.at[pl.ds(b0, cc)], msgs, sem3)
        pltpu.sync_copy(src_hbm.at[pl.ds(b0, cc)], src_v)
        pltpu.sync_copy(dst_hbm.at[pl.ds(b0, cc)], dst_v)
        cp1 = pltpu.async_copy(vc_hbm.at[dst_v], vcrows, sem1)
        cp2 = pltpu.async_copy(bh_hbm.at[src_v], brows, sem2)
        cps.wait()
        cp1.wait()
        cp2.wait()

        def edge_body(j, carry2):
            for k in range(8):
                sl = pl.ds(k * 16, 16)
                msgs[j, sl] = msgs[j, sl] * vcrows[j, sl]
                brows[j, sl] = (brows[j, sl]
                                + vcrows[j, pl.ds(128 + k * 16, 16)])
            return carry2

        lax.fori_loop(0, cc, edge_body, 0)
        pltpu.sync_copy(brows, g_out.at[pl.ds(b0, cc)])
        pltpu.sync_copy(msgs, agg_sh.at[src_v], add=True)
        return carry

    lax.fori_loop(0, nw, chunk_body, 0)
    plsc.subcore_barrier()
    pltpu.sync_copy(agg_sh.at[pl.ds(r0, rows)], agg_out.at[c, pl.ds(r0, rows)])


def _sc_cnt_body(src_hbm, zero_hbm, cnt_out,
                 src_v, ones_v, cnt_sh, *, n_pad, nbase, nrem, cc):
    c = lax.axis_index("c")
    s = lax.axis_index("s")
    w = c * _NS + s
    nw = nbase + jnp.where(w < nrem, 1, 0)
    rows = n_pad // _NS
    r0 = s * rows

    pltpu.sync_copy(zero_hbm.at[pl.ds(r0, rows)], cnt_sh.at[pl.ds(r0, rows)])
    for j in range(cc):
        for k in range(8):
            ones_v[j, pl.ds(k * 16, 16)] = jnp.full((16,), 1.0, _F32)
    plsc.subcore_barrier()

    def chunk_body(ci, carry):
        b0 = pl.multiple_of((ci * _NW + w) * cc, 8)
        pltpu.sync_copy(src_hbm.at[pl.ds(b0, cc)], src_v)
        pltpu.sync_copy(ones_v, cnt_sh.at[src_v], add=True)
        return carry

    lax.fori_loop(0, nw, chunk_body, 0)
    plsc.subcore_barrier()
    pltpu.sync_copy(cnt_sh.at[pl.ds(r0, rows)], cnt_out.at[c, pl.ds(r0, rows)])


def _sc_l2_body(src_hbm, dst_hbm, b2_hbm, c2_hbm, g_out,
                src_v, dst_v, brows, crows, sem1, sem2, *, nbase, nrem, cc):
    c = lax.axis_index("c")
    s = lax.axis_index("s")
    w = c * _NS + s
    nw = nbase + jnp.where(w < nrem, 1, 0)

    def chunk_body(ci, carry):
        b0 = pl.multiple_of((ci * _NW + w) * cc, 8)
        pltpu.sync_copy(src_hbm.at[pl.ds(b0, cc)], src_v)
        pltpu.sync_copy(dst_hbm.at[pl.ds(b0, cc)], dst_v)
        cp1 = pltpu.async_copy(b2_hbm.at[src_v], brows, sem1)
        cp2 = pltpu.async_copy(c2_hbm.at[dst_v], crows, sem2)
        cp1.wait()
        cp2.wait()

        def edge_body(j, carry2):
            for k in range(8):
                sl = pl.ds(k * 16, 16)
                brows[j, sl] = brows[j, sl] + crows[j, sl]
            return carry2

        lax.fori_loop(0, cc, edge_body, 0)
        pltpu.sync_copy(brows, g_out.at[pl.ds(b0, cc)])
        return carry

    lax.fori_loop(0, nw, chunk_body, 0)


def _full_spec(shape):
    return pl.BlockSpec(shape, lambda i: tuple(0 for _ in shape))


def _row2(w, b):
    return jnp.stack([w.reshape(-1), b.reshape(-1)], axis=0)


def kernel(x, edge_attr, edge_index, params):
    n = x.shape[0]
    e = edge_attr.shape[0]
    d = 128
    ca, cb = 80, 128
    assert e % ca == 0 and e % cb == 0
    n_pad = -(-n // 128) * 128
    nbase_a, nrem_a = (e // ca) // _NW, (e // ca) % _NW
    nbase_b, nrem_b = (e // cb) // _NW, (e // cb) % _NW
    be = 4000
    grid_e = e // be
    inv_e = 1.0 / e

    src = edge_index[0]
    dst = edge_index[1]
    ea = edge_attr.reshape(e)
    p = params
    l1, l2 = p["layers"][0], p["layers"][1]
    mlp = p["mlp"]

    f32 = jnp.float32
    sds = jax.ShapeDtypeStruct

    node1 = pl.pallas_call(
        _node_stage1_body,
        grid=(1,),
        in_specs=[_full_spec((n, 2))] + [_full_spec(s) for s in
                  [(2, d), (1, d), (d, d), (1, d), (d, d), (1, d),
                   (d, d), (1, d), (d, d), (1, d)]],
        out_specs=[_full_spec((n, d)), _full_spec((n, d)),
                   _full_spec((n, 256)), _full_spec((n, d))],
        out_shape=[sds((n, d), f32), sds((n, d), f32), sds((n, 256), f32),
                   sds((n, d), f32)],
    )
    h0, u1h, vc1, b1h = node1(
        x, p["h_proj"]["W"], p["h_proj"]["b"].reshape(1, d),
        l1["U"]["W"], l1["U"]["b"].reshape(1, d),
        l1["V"]["W"], l1["V"]["b"].reshape(1, d),
        l1["B"]["W"], l1["B"]["b"].reshape(1, d),
        l1["C"]["W"], l1["C"]["b"].reshape(1, d))

    wp2 = _row2(p["e_proj"]["W"], p["e_proj"]["b"])
    be_s = 8000
    ea2 = ea.reshape(e, 1)
    sgate = pl.pallas_call(
        _sgate_body,
        grid=(e // be_s,),
        in_specs=[pl.BlockSpec((be_s, 1), lambda i: (i, 0)),
                  _full_spec((2, d))],
        out_specs=pl.BlockSpec((be_s, d), lambda i: (i, 0)),
        out_shape=sds((e, d), f32),
    )
    sgv = sgate(ea2, wp2)

    mesh = plsc.VectorSubcoreMesh(core_axis_name="c", subcore_axis_name="s")
    sc1 = pl.kernel(
        functools.partial(_sc_l1_body, n_pad=n_pad, nbase=nbase_a, nrem=nrem_a, cc=ca),
        out_type=(sds((e, d), f32), sds((2, n_pad, d), f32)),
        mesh=mesh,
        scratch_types=[
            pltpu.VMEM((ca,), jnp.int32),
            pltpu.VMEM((ca,), jnp.int32),
            pltpu.VMEM((ca, 256), f32),
            pltpu.VMEM((ca, d), f32),
            pltpu.VMEM((ca, d), f32),
            pltpu.VMEM_SHARED((n_pad, d), f32),
            pltpu.SemaphoreType.DMA,
            pltpu.SemaphoreType.DMA,
            pltpu.SemaphoreType.DMA,
        ],
    )
    g1, agg2 = sc1(sgv, src, dst, vc1, b1h, jnp.zeros((n_pad, d), f32))

    sc_cnt = pl.kernel(
        functools.partial(_sc_cnt_body, n_pad=n_pad, nbase=nbase_b, nrem=nrem_b, cc=cb),
        out_type=sds((2, n_pad, d), f32),
        mesh=mesh,
        scratch_types=[
            pltpu.VMEM((cb,), jnp.int32),
            pltpu.VMEM((cb, d), f32),
            pltpu.VMEM_SHARED((n_pad, d), f32),
        ],
    )
    cntf = sc_cnt(src, jnp.zeros((n_pad, d), f32))

    ea_spec = pl.BlockSpec((be, 1), lambda i: (i, 0))
    g_spec = pl.BlockSpec((be, d), lambda i: (i, 0))
    sums_spec = pl.BlockSpec((8, d), lambda i: (0, 0))
    sums1 = pl.pallas_call(
        _epass1_l1_body,
        grid=(grid_e,),
        in_specs=[ea_spec, g_spec, _full_spec((2, d)), _full_spec((d, d)),
                  _full_spec((1, d))],
        out_specs=sums_spec,
        out_shape=sds((8, d), f32),
    )(ea2, g1, wp2, l1["A"]["W"], l1["A"]["b"].reshape(1, d))

    node2 = pl.pallas_call(
        functools.partial(_node_stage2_body, n=n),
        grid=(1,),
        in_specs=[_full_spec((n, d)), _full_spec((2, n_pad, d)),
                  _full_spec((2, n_pad, d)), _full_spec((n, d)),
                  _full_spec((1, d)), _full_spec((1, d)),
                  _full_spec((d, d)), _full_spec((1, d)),
                  _full_spec((d, d)), _full_spec((1, d))],
        out_specs=[_full_spec((n, d)), _full_spec((n, d))],
        out_shape=[sds((n, d), f32), sds((n, d), f32)],
    )
    b2h, c2h = node2(
        u1h, agg2, cntf, h0,
        l1["h_bn_g"].reshape(1, d), l1["h_bn_b"].reshape(1, d),
        l2["B"]["W"], l2["B"]["b"].reshape(1, d),
        l2["C"]["W"], l2["C"]["b"].reshape(1, d))

    sc2 = pl.kernel(
        functools.partial(_sc_l2_body, nbase=nbase_b, nrem=nrem_b, cc=cb),
        out_type=sds((e, d), f32),
        mesh=mesh,
        scratch_types=[
            pltpu.VMEM((cb,), jnp.int32),
            pltpu.VMEM((cb,), jnp.int32),
            pltpu.VMEM((cb, d), f32),
            pltpu.VMEM((cb, d), f32),
            pltpu.SemaphoreType.DMA,
            pltpu.SemaphoreType.DMA,
        ],
    )
    g2 = sc2(src, dst, b2h, c2h)

    bn1 = _row2(l1["e_bn_g"], l1["e_bn_b"])
    bn2 = _row2(l2["e_bn_g"], l2["e_bn_b"])
    sums2 = pl.pallas_call(
        functools.partial(_epass1_l2_body, inv_e=inv_e),
        grid=(grid_e,),
        in_specs=[ea_spec, g_spec, g_spec, _full_spec((2, d)),
                  _full_spec((d, d)), _full_spec((1, d)), _full_spec((8, d)),
                  _full_spec((2, d)), _full_spec((d, d)), _full_spec((1, d))],
        out_specs=sums_spec,
        out_shape=sds((8, d), f32),
    )(ea2, g1, g2, wp2, l1["A"]["W"], l1["A"]["b"].reshape(1, d), sums1,
      bn1, l2["A"]["W"], l2["A"]["b"].reshape(1, d))

    z = pl.pallas_call(
        functools.partial(_final_body, inv_e=inv_e),
        grid=(grid_e,),
        in_specs=[ea_spec, g_spec, g_spec, _full_spec((2, d)),
                  _full_spec((d, d)), _full_spec((1, d)), _full_spec((8, d)),
                  _full_spec((2, d)), _full_spec((d, d)), _full_spec((1, d)),
                  _full_spec((8, d)), _full_spec((2, d)),
                  _full_spec((d, d)), _full_spec((1, d)),
                  _full_spec((d, d)), _full_spec((1, d)),
                  _full_spec((d, 1)), _full_spec((1, 1))],
        out_specs=pl.BlockSpec((be, 1), lambda i: (i, 0)),
        out_shape=sds((e, 1), f32),
    )(ea2, g1, g2, wp2, l1["A"]["W"], l1["A"]["b"].reshape(1, d), sums1,
      bn1, l2["A"]["W"], l2["A"]["b"].reshape(1, d), sums2, bn2,
      mlp[0]["W"], mlp[0]["b"].reshape(1, d),
      mlp[1]["W"], mlp[1]["b"].reshape(1, d),
      mlp[2]["W"], mlp[2]["b"].reshape(1, 1))
    return z


# speedup vs baseline: 2.3315x; 1.0180x over previous
import functools

import jax
import jax.numpy as jnp
from jax import lax
from jax.experimental import pallas as pl
from jax.experimental.pallas import tpu as pltpu
from jax.experimental.pallas import tpu_sc as plsc

_NC = 2
_NS = 16
_NW = _NC * _NS
_C = 128
_EPS = 1e-5
_F32 = jnp.float32


def _relu(v):
    return jnp.maximum(v, 0.0)


def _sigmoid(v):
    return 1.0 / (1.0 + jnp.exp(-v))


def _dot(a, b):
    return jnp.dot(a, b, preferred_element_type=jnp.float32)


def _node_stage1_body(x_ref, wh_ref, bh_ref, u_ref, bu_ref, v_ref, bv_ref,
                      b_ref, bb_ref, c_ref, bc_ref,
                      h0_ref, uh_ref, vc_ref, bh_out_ref):
    x = x_ref[...]
    h0 = _relu(x[:, 0:1] * wh_ref[0:1, :] + x[:, 1:2] * wh_ref[1:2, :]
               + bh_ref[0:1, :])
    h0_ref[...] = h0
    uh_ref[...] = _dot(h0, u_ref[...]) + bu_ref[0:1, :]
    vc_ref[:, 0:128] = _dot(h0, v_ref[...]) + bv_ref[0:1, :]
    vc_ref[:, 128:256] = _dot(h0, c_ref[...]) + bc_ref[0:1, :]
    bh_out_ref[...] = _dot(h0, b_ref[...]) + bb_ref[0:1, :]


def _node_stage2_body(uh_ref, agg2_ref, cntf_ref, h0_ref, g_ref, b_ref,
                      b2_ref, bb2_ref, c2_ref, bc2_ref,
                      b2h_ref, c2h_ref, *, n):
    agg = agg2_ref[0, 0:n, :] + agg2_ref[1, 0:n, :]
    cnt = cntf_ref[0, 0:n, 0:1] + cntf_ref[1, 0:n, 0:1]
    q = uh_ref[...] + agg / jnp.maximum(cnt, 1.0)
    m = jnp.mean(q, axis=0, keepdims=True)
    v = jnp.mean((q - m) ** 2, axis=0, keepdims=True)
    h1 = h0_ref[...] + _relu((q - m) * lax.rsqrt(v + _EPS) * g_ref[0:1, :]
                             + b_ref[0:1, :])
    b2h_ref[...] = _dot(h1, b2_ref[...]) + bb2_ref[0:1, :]
    c2h_ref[...] = _dot(h1, c2_ref[...]) + bc2_ref[0:1, :]


def _e0_block(ea, wp_ref):
    return _relu(ea * wp_ref[0:1, :] + wp_ref[1:2, :])


def _sgate_body(ea_ref, wp_ref, s_ref):
    s_ref[...] = _sigmoid(_e0_block(ea_ref[...], wp_ref))


def _epass1_l1_body(ea_ref, g1_ref, wp_ref, a1_ref, ba1_ref, sums_ref):
    i = pl.program_id(0)
    e0 = _e0_block(ea_ref[...], wp_ref)
    y = _dot(e0, a1_ref[...]) + ba1_ref[0:1, :] + g1_ref[...]

    @pl.when(i == 0)
    def _():
        sums_ref[...] = jnp.zeros_like(sums_ref)

    sums_ref[0:1, :] += jnp.sum(y, axis=0, keepdims=True)
    sums_ref[1:2, :] += jnp.sum(y * y, axis=0, keepdims=True)


def _e1_block(ea, g1, wp_ref, a1_ref, ba1_ref, sums1_ref, bn1_ref, inv_e):
    e0 = _e0_block(ea, wp_ref)
    y1 = _dot(e0, a1_ref[...]) + ba1_ref[0:1, :] + g1
    m1 = sums1_ref[0:1, :] * inv_e
    v1 = sums1_ref[1:2, :] * inv_e - m1 * m1
    return e0 + _relu((y1 - m1) * lax.rsqrt(v1 + _EPS) * bn1_ref[0:1, :]
                      + bn1_ref[1:2, :])


def _epass1_l2_body(ea_ref, g1_ref, g2_ref, wp_ref, a1_ref, ba1_ref,
                    sums1_ref, bn1_ref, a2_ref, ba2_ref, sums2_ref, *,
                    inv_e):
    i = pl.program_id(0)
    e1 = _e1_block(ea_ref[...], g1_ref[...], wp_ref, a1_ref, ba1_ref,
                   sums1_ref, bn1_ref, inv_e)
    y2 = _dot(e1, a2_ref[...]) + ba2_ref[0:1, :] + g2_ref[...]

    @pl.when(i == 0)
    def _():
        sums2_ref[...] = jnp.zeros_like(sums2_ref)

    sums2_ref[0:1, :] += jnp.sum(y2, axis=0, keepdims=True)
    sums2_ref[1:2, :] += jnp.sum(y2 * y2, axis=0, keepdims=True)


def _final_body(ea_ref, g1_ref, g2_ref, wp_ref, a1_ref, ba1_ref, sums1_ref,
                bn1_ref, a2_ref, ba2_ref, sums2_ref, bn2_ref,
                w1_ref, b1_ref, w2_ref, b2_ref, w3_ref, b3_ref, z_ref, *,
                inv_e):
    e1 = _e1_block(ea_ref[...], g1_ref[...], wp_ref, a1_ref, ba1_ref,
                   sums1_ref, bn1_ref, inv_e)
    y2 = _dot(e1, a2_ref[...]) + ba2_ref[0:1, :] + g2_ref[...]
    m2 = sums2_ref[0:1, :] * inv_e
    v2 = sums2_ref[1:2, :] * inv_e - m2 * m2
    e2 = e1 + _relu((y2 - m2) * lax.rsqrt(v2 + _EPS) * bn2_ref[0:1, :]
                    + bn2_ref[1:2, :])
    t = _dot(e2, w1_ref[...]) + b1_ref[0:1, :]
    t = t * _sigmoid(t)
    t = _dot(t, w2_ref[...]) + b2_ref[0:1, :]
    t = t * _sigmoid(t)
    z_ref[...] = _sigmoid(_dot(t, w3_ref[...]) + b3_ref[0:1, :])


def _sc_l1_body(s_hbm, src_hbm, dst_hbm, vc_hbm, bh_hbm,
                zero_hbm,
                g_out, agg_out,
                src_v, dst_v, vcrows, brows, msgs,
                agg_sh, sem1, sem2, sem3, sem4, sem5, *, n_pad, nbase,
                nrem, cc):
    c = lax.axis_index("c")
    s = lax.axis_index("s")
    w = c * _NS + s
    nw = nbase + jnp.where(w < nrem, 1, 0)
    rows = n_pad // _NS
    r0 = s * rows

    pltpu.sync_copy(zero_hbm.at[pl.ds(r0, rows)], agg_sh.at[pl.ds(r0, rows)])
    plsc.subcore_barrier()

    def chunk_body(ci, carry):
        b0 = pl.multiple_of((ci * _NW + w) * cc, 8)

        @pl.when(ci > 0)
        def _():
            pltpu.make_async_copy(s_hbm.at[pl.ds(0, cc)], msgs, sem5).wait()

        cps = pltpu.async_copy(s_hbm.at[pl.ds(b0, cc)], msgs, sem3)
        pltpu.sync_copy(src_hbm.at[pl.ds(b0, cc)], src_v)
        pltpu.sync_copy(dst_hbm.at[pl.ds(b0, cc)], dst_v)
        cp1 = pltpu.async_copy(vc_hbm.at[dst_v], vcrows, sem1)

        @pl.when(ci > 0)
        def _():
            pltpu.make_async_copy(s_hbm.at[pl.ds(0, cc)], brows, sem4).wait()

        cp2 = pltpu.async_copy(bh_hbm.at[src_v], brows, sem2)
        cps.wait()
        cp1.wait()
        cp2.wait()

        def edge_body(j, carry2):
            for k in range(8):
                sl = pl.ds(k * 16, 16)
                msgs[j, sl] = msgs[j, sl] * vcrows[j, sl]
                brows[j, sl] = (brows[j, sl]
                                + vcrows[j, pl.ds(128 + k * 16, 16)])
            return carry2

        lax.fori_loop(0, cc, edge_body, 0)
        pltpu.async_copy(brows, g_out.at[pl.ds(b0, cc)], sem4)
        pltpu.async_copy(msgs, agg_sh.at[src_v], sem5, add=True)
        return carry

    lax.fori_loop(0, nw, chunk_body, 0)
    pltpu.make_async_copy(s_hbm.at[pl.ds(0, cc)], msgs, sem5).wait()
    pltpu.make_async_copy(s_hbm.at[pl.ds(0, cc)], brows, sem4).wait()
    plsc.subcore_barrier()
    pltpu.sync_copy(agg_sh.at[pl.ds(r0, rows)], agg_out.at[c, pl.ds(r0, rows)])


def _sc_cnt_body(src_hbm, zero_hbm, cnt_out,
                 src_v, ones_v, cnt_sh, *, n_pad, nbase, nrem, cc):
    c = lax.axis_index("c")
    s = lax.axis_index("s")
    w = c * _NS + s
    nw = nbase + jnp.where(w < nrem, 1, 0)
    rows = n_pad // _NS
    r0 = s * rows

    pltpu.sync_copy(zero_hbm.at[pl.ds(r0, rows)], cnt_sh.at[pl.ds(r0, rows)])
    for j in range(cc):
        for k in range(8):
            ones_v[j, pl.ds(k * 16, 16)] = jnp.full((16,), 1.0, _F32)
    plsc.subcore_barrier()

    def chunk_body(ci, carry):
        b0 = pl.multiple_of((ci * _NW + w) * cc, 8)
        pltpu.sync_copy(src_hbm.at[pl.ds(b0, cc)], src_v)
        pltpu.sync_copy(ones_v, cnt_sh.at[src_v], add=True)
        return carry

    lax.fori_loop(0, nw, chunk_body, 0)
    plsc.subcore_barrier()
    pltpu.sync_copy(cnt_sh.at[pl.ds(r0, rows)], cnt_out.at[c, pl.ds(r0, rows)])


def _sc_l2_body(src_hbm, dst_hbm, b2_hbm, c2_hbm, g_out,
                src_v, dst_v, brows, crows, sem1, sem2, *, nbase, nrem, cc):
    c = lax.axis_index("c")
    s = lax.axis_index("s")
    w = c * _NS + s
    nw = nbase + jnp.where(w < nrem, 1, 0)

    def chunk_body(ci, carry):
        b0 = pl.multiple_of((ci * _NW + w) * cc, 8)
        pltpu.sync_copy(src_hbm.at[pl.ds(b0, cc)], src_v)
        pltpu.sync_copy(dst_hbm.at[pl.ds(b0, cc)], dst_v)
        cp1 = pltpu.async_copy(b2_hbm.at[src_v], brows, sem1)
        cp2 = pltpu.async_copy(c2_hbm.at[dst_v], crows, sem2)
        cp1.wait()
        cp2.wait()

        def edge_body(j, carry2):
            for k in range(8):
                sl = pl.ds(k * 16, 16)
                brows[j, sl] = brows[j, sl] + crows[j, sl]
            return carry2

        lax.fori_loop(0, cc, edge_body, 0)
        pltpu.sync_copy(brows, g_out.at[pl.ds(b0, cc)])
        return carry

    lax.fori_loop(0, nw, chunk_body, 0)


def _full_spec(shape):
    return pl.BlockSpec(shape, lambda i: tuple(0 for _ in shape))


def _row2(w, b):
    return jnp.stack([w.reshape(-1), b.reshape(-1)], axis=0)


def kernel(x, edge_attr, edge_index, params):
    n = x.shape[0]
    e = edge_attr.shape[0]
    d = 128
    ca, cb = 80, 128
    assert e % ca == 0 and e % cb == 0
    n_pad = -(-n // 128) * 128
    nbase_a, nrem_a = (e // ca) // _NW, (e // ca) % _NW
    nbase_b, nrem_b = (e // cb) // _NW, (e // cb) % _NW
    be = 4000
    grid_e = e // be
    inv_e = 1.0 / e

    src = edge_index[0]
    dst = edge_index[1]
    ea = edge_attr.reshape(e)
    p = params
    l1, l2 = p["layers"][0], p["layers"][1]
    mlp = p["mlp"]

    f32 = jnp.float32
    sds = jax.ShapeDtypeStruct

    node1 = pl.pallas_call(
        _node_stage1_body,
        grid=(1,),
        in_specs=[_full_spec((n, 2))] + [_full_spec(s) for s in
                  [(2, d), (1, d), (d, d), (1, d), (d, d), (1, d),
                   (d, d), (1, d), (d, d), (1, d)]],
        out_specs=[_full_spec((n, d)), _full_spec((n, d)),
                   _full_spec((n, 256)), _full_spec((n, d))],
        out_shape=[sds((n, d), f32), sds((n, d), f32), sds((n, 256), f32),
                   sds((n, d), f32)],
    )
    h0, u1h, vc1, b1h = node1(
        x, p["h_proj"]["W"], p["h_proj"]["b"].reshape(1, d),
        l1["U"]["W"], l1["U"]["b"].reshape(1, d),
        l1["V"]["W"], l1["V"]["b"].reshape(1, d),
        l1["B"]["W"], l1["B"]["b"].reshape(1, d),
        l1["C"]["W"], l1["C"]["b"].reshape(1, d))

    wp2 = _row2(p["e_proj"]["W"], p["e_proj"]["b"])
    be_s = 8000
    ea2 = ea.reshape(e, 1)
    sgate = pl.pallas_call(
        _sgate_body,
        grid=(e // be_s,),
        in_specs=[pl.BlockSpec((be_s, 1), lambda i: (i, 0)),
                  _full_spec((2, d))],
        out_specs=pl.BlockSpec((be_s, d), lambda i: (i, 0)),
        out_shape=sds((e, d), f32),
    )
    sgv = sgate(ea2, wp2)

    mesh = plsc.VectorSubcoreMesh(core_axis_name="c", subcore_axis_name="s")
    sc1 = pl.kernel(
        functools.partial(_sc_l1_body, n_pad=n_pad, nbase=nbase_a, nrem=nrem_a, cc=ca),
        out_type=(sds((e, d), f32), sds((2, n_pad, d), f32)),
        mesh=mesh,
        scratch_types=[
            pltpu.VMEM((ca,), jnp.int32),
            pltpu.VMEM((ca,), jnp.int32),
            pltpu.VMEM((ca, 256), f32),
            pltpu.VMEM((ca, d), f32),
            pltpu.VMEM((ca, d), f32),
            pltpu.VMEM_SHARED((n_pad, d), f32),
            pltpu.SemaphoreType.DMA,
            pltpu.SemaphoreType.DMA,
            pltpu.SemaphoreType.DMA,
            pltpu.SemaphoreType.DMA,
            pltpu.SemaphoreType.DMA,
        ],
    )
    g1, agg2 = sc1(sgv, src, dst, vc1, b1h, jnp.zeros((n_pad, d), f32))

    sc_cnt = pl.kernel(
        functools.partial(_sc_cnt_body, n_pad=n_pad, nbase=nbase_b, nrem=nrem_b, cc=cb),
        out_type=sds((2, n_pad, d), f32),
        mesh=mesh,
        scratch_types=[
            pltpu.VMEM((cb,), jnp.int32),
            pltpu.VMEM((cb, d), f32),
            pltpu.VMEM_SHARED((n_pad, d), f32),
        ],
    )
    cntf = sc_cnt(src, jnp.zeros((n_pad, d), f32))

    ea_spec = pl.BlockSpec((be, 1), lambda i: (i, 0))
    g_spec = pl.BlockSpec((be, d), lambda i: (i, 0))
    sums_spec = pl.BlockSpec((8, d), lambda i: (0, 0))
    sums1 = pl.pallas_call(
        _epass1_l1_body,
        grid=(grid_e,),
        in_specs=[ea_spec, g_spec, _full_spec((2, d)), _full_spec((d, d)),
                  _full_spec((1, d))],
        out_specs=sums_spec,
        out_shape=sds((8, d), f32),
    )(ea2, g1, wp2, l1["A"]["W"], l1["A"]["b"].reshape(1, d))

    node2 = pl.pallas_call(
        functools.partial(_node_stage2_body, n=n),
        grid=(1,),
        in_specs=[_full_spec((n, d)), _full_spec((2, n_pad, d)),
                  _full_spec((2, n_pad, d)), _full_spec((n, d)),
                  _full_spec((1, d)), _full_spec((1, d)),
                  _full_spec((d, d)), _full_spec((1, d)),
                  _full_spec((d, d)), _full_spec((1, d))],
        out_specs=[_full_spec((n, d)), _full_spec((n, d))],
        out_shape=[sds((n, d), f32), sds((n, d), f32)],
    )
    b2h, c2h = node2(
        u1h, agg2, cntf, h0,
        l1["h_bn_g"].reshape(1, d), l1["h_bn_b"].reshape(1, d),
        l2["B"]["W"], l2["B"]["b"].reshape(1, d),
        l2["C"]["W"], l2["C"]["b"].reshape(1, d))

    sc2 = pl.kernel(
        functools.partial(_sc_l2_body, nbase=nbase_b, nrem=nrem_b, cc=cb),
        out_type=sds((e, d), f32),
        mesh=mesh,
        scratch_types=[
            pltpu.VMEM((cb,), jnp.int32),
            pltpu.VMEM((cb,), jnp.int32),
            pltpu.VMEM((cb, d), f32),
            pltpu.VMEM((cb, d), f32),
            pltpu.SemaphoreType.DMA,
            pltpu.SemaphoreType.DMA,
        ],
    )
    g2 = sc2(src, dst, b2h, c2h)

    bn1 = _row2(l1["e_bn_g"], l1["e_bn_b"])
    bn2 = _row2(l2["e_bn_g"], l2["e_bn_b"])
    sums2 = pl.pallas_call(
        functools.partial(_epass1_l2_body, inv_e=inv_e),
        grid=(grid_e,),
        in_specs=[ea_spec, g_spec, g_spec, _full_spec((2, d)),
                  _full_spec((d, d)), _full_spec((1, d)), _full_spec((8, d)),
                  _full_spec((2, d)), _full_spec((d, d)), _full_spec((1, d))],
        out_specs=sums_spec,
        out_shape=sds((8, d), f32),
    )(ea2, g1, g2, wp2, l1["A"]["W"], l1["A"]["b"].reshape(1, d), sums1,
      bn1, l2["A"]["W"], l2["A"]["b"].reshape(1, d))

    z = pl.pallas_call(
        functools.partial(_final_body, inv_e=inv_e),
        grid=(grid_e,),
        in_specs=[ea_spec, g_spec, g_spec, _full_spec((2, d)),
                  _full_spec((d, d)), _full_spec((1, d)), _full_spec((8, d)),
                  _full_spec((2, d)), _full_spec((d, d)), _full_spec((1, d)),
                  _full_spec((8, d)), _full_spec((2, d)),
                  _full_spec((d, d)), _full_spec((1, d)),
                  _full_spec((d, d)), _full_spec((1, d)),
                  _full_spec((d, 1)), _full_spec((1, 1))],
        out_specs=pl.BlockSpec((be, 1), lambda i: (i, 0)),
        out_shape=sds((e, 1), f32),
    )(ea2, g1, g2, wp2, l1["A"]["W"], l1["A"]["b"].reshape(1, d), sums1,
      bn1, l2["A"]["W"], l2["A"]["b"].reshape(1, d), sums2, bn2,
      mlp[0]["W"], mlp[0]["b"].reshape(1, d),
      mlp[1]["W"], mlp[1]["b"].reshape(1, d),
      mlp[2]["W"], mlp[2]["b"].reshape(1, 1))
    return z
